# Initial kernel scaffold; baseline (speedup 1.0000x reference)
import jax
import jax.numpy as jnp
from jax.experimental import pallas as pl


def kernel(x, edge_index, W_in, b_in, W1, al1, ar1, b1, g1, be1, W2, al2, ar2, b2, g2, be2, Wr0, br0, Wr1, br1, Wr2, br2):
    raise NotImplementedError("write your pallas kernel here")


# speedup vs baseline: 49.9354x; 49.9354x over previous
import functools

import jax
import jax.numpy as jnp
from jax import lax
from jax.experimental import pallas as pl
from jax.experimental.pallas import tpu as pltpu

_N_BLK = 2000


def _elu(x):
    return jnp.where(x > 0, x, jnp.exp(x) - 1.0)


def _lrelu(x):
    return jnp.maximum(x, 0.2 * x)


def _body_a(x_r, win_r, bin_r, w1_r, al_r, ar_r,
            h_r, tbl_r, er_r, mel_r, mer_r):
    i = pl.program_id(0)
    h = jnp.dot(x_r[...], win_r[...], preferred_element_type=jnp.float32) + bin_r[...]
    feat = jnp.dot(h, w1_r[...], preferred_element_type=jnp.float32)
    el = jnp.dot(feat, al_r[...], preferred_element_type=jnp.float32)
    er = jnp.dot(feat, ar_r[...], preferred_element_type=jnp.float32)
    h_r[...] = h
    tbl_r[:, :128] = feat
    tbl_r[:, 128:144] = el
    er_r[...] = er

    @pl.when(i == 0)
    def _():
        mel_r[...] = jnp.full((1, 16), -1e30, jnp.float32)
        mer_r[...] = jnp.full((1, 16), -1e30, jnp.float32)

    mel_r[...] = jnp.maximum(mel_r[...], jnp.max(el, axis=0, keepdims=True))
    mer_r[...] = jnp.maximum(mer_r[...], jnp.max(er, axis=0, keepdims=True))


def _stage_a(x, w_in, b_in, w1, al_blk, ar_blk):
    n = x.shape[0]
    grid = n // _N_BLK
    return pl.pallas_call(
        _body_a,
        grid=(grid,),
        in_specs=[
            pl.BlockSpec((_N_BLK, 128), lambda i: (i, 0)),
            pl.BlockSpec((128, 128), lambda i: (0, 0)),
            pl.BlockSpec((1, 128), lambda i: (0, 0)),
            pl.BlockSpec((128, 128), lambda i: (0, 0)),
            pl.BlockSpec((128, 16), lambda i: (0, 0)),
            pl.BlockSpec((128, 16), lambda i: (0, 0)),
        ],
        out_specs=[
            pl.BlockSpec((_N_BLK, 128), lambda i: (i, 0)),
            pl.BlockSpec((_N_BLK, 144), lambda i: (i, 0)),
            pl.BlockSpec((_N_BLK, 16), lambda i: (i, 0)),
            pl.BlockSpec((1, 16), lambda i: (0, 0)),
            pl.BlockSpec((1, 16), lambda i: (0, 0)),
        ],
        out_shape=[
            jax.ShapeDtypeStruct((n, 128), jnp.float32),
            jax.ShapeDtypeStruct((n, 144), jnp.float32),
            jax.ShapeDtypeStruct((n, 16), jnp.float32),
            jax.ShapeDtypeStruct((1, 16), jnp.float32),
            jax.ShapeDtypeStruct((1, 16), jnp.float32),
        ],
    )(x, w_in, b_in, w1, al_blk, ar_blk)


def _body_b(acc_r, h_r, denp_r, b1_r, g1_r, be1_r, w2_r, al2_r, ar2_r,
            h2_r, tbl2_r, er2_r, mel2_r, mer2_r, stats_r):
    p = pl.program_id(0)
    i = pl.program_id(1)
    asum = acc_r[0] + acc_r[1]
    den = jnp.dot(asum[:, 128:136], denp_r[...],
                  preferred_element_type=jnp.float32)
    den = jnp.where(den == 0.0, 1.0, den)
    rst = asum[:, :128] / den + b1_r[...]

    @pl.when(p == 0)
    def _():
        @pl.when(i == 0)
        def _():
            stats_r[...] = jnp.zeros_like(stats_r)

        stats_r[0:1] += jnp.sum(rst, axis=0, keepdims=True)
        stats_r[1:2] += jnp.sum(rst * rst, axis=0, keepdims=True)

    @pl.when(p == 1)
    def _():
        n_total = pl.num_programs(1) * rst.shape[0]
        mu = stats_r[0:1] / n_total
        var = stats_r[1:2] / n_total - mu * mu
        hbn = (rst - mu) * lax.rsqrt(var + 1e-5) * g1_r[...] + be1_r[...]
        h2 = h_r[...] + _elu(hbn)
        h2_r[...] = h2
        feat2 = jnp.dot(h2, w2_r[...], preferred_element_type=jnp.float32)
        el2 = jnp.dot(feat2, al2_r[...], preferred_element_type=jnp.float32)
        er2 = jnp.dot(feat2, ar2_r[...], preferred_element_type=jnp.float32)
        tbl2_r[:, :128] = feat2
        tbl2_r[:, 128:144] = el2
        er2_r[...] = er2

        @pl.when(i == 0)
        def _():
            mel2_r[...] = jnp.full((1, 16), -1e30, jnp.float32)
            mer2_r[...] = jnp.full((1, 16), -1e30, jnp.float32)

        mel2_r[...] = jnp.maximum(mel2_r[...], jnp.max(el2, axis=0, keepdims=True))
        mer2_r[...] = jnp.maximum(mer2_r[...], jnp.max(er2, axis=0, keepdims=True))


def _stage_b(acc, h, denp, b1, g1, be1, w2, al2_blk, ar2_blk):
    n = h.shape[0]
    grid = n // _N_BLK
    return pl.pallas_call(
        _body_b,
        grid=(2, grid),
        in_specs=[
            pl.BlockSpec((2, _N_BLK, 144), lambda p, i: (0, i, 0)),
            pl.BlockSpec((_N_BLK, 128), lambda p, i: (i, 0)),
            pl.BlockSpec((8, 128), lambda p, i: (0, 0)),
            pl.BlockSpec((1, 128), lambda p, i: (0, 0)),
            pl.BlockSpec((1, 128), lambda p, i: (0, 0)),
            pl.BlockSpec((1, 128), lambda p, i: (0, 0)),
            pl.BlockSpec((128, 128), lambda p, i: (0, 0)),
            pl.BlockSpec((128, 16), lambda p, i: (0, 0)),
            pl.BlockSpec((128, 16), lambda p, i: (0, 0)),
        ],
        out_specs=[
            pl.BlockSpec((_N_BLK, 128), lambda p, i: (i, 0)),
            pl.BlockSpec((_N_BLK, 144), lambda p, i: (i, 0)),
            pl.BlockSpec((_N_BLK, 16), lambda p, i: (i, 0)),
            pl.BlockSpec((1, 16), lambda p, i: (0, 0)),
            pl.BlockSpec((1, 16), lambda p, i: (0, 0)),
        ],
        out_shape=[
            jax.ShapeDtypeStruct((n, 128), jnp.float32),
            jax.ShapeDtypeStruct((n, 144), jnp.float32),
            jax.ShapeDtypeStruct((n, 16), jnp.float32),
            jax.ShapeDtypeStruct((1, 16), jnp.float32),
            jax.ShapeDtypeStruct((1, 16), jnp.float32),
        ],
        scratch_shapes=[pltpu.VMEM((2, 128), jnp.float32)],
    )(acc, h, denp, b1, g1, be1, w2, al2_blk, ar2_blk)


def _body_c(acc_r, h2_r, b2_r, g2_r, be2_r,
            wr0_r, br0_r, wr1_r, br1_r, wr2_r, br2_r,
            y_r, stats_r, hsum_r):
    p = pl.program_id(0)
    i = pl.program_id(1)
    asum = acc_r[0] + acc_r[1]
    den = lax.broadcast_in_dim(asum[:, 128:129], asum[:, :128].shape, (0, 1))
    den = jnp.where(den == 0.0, 1.0, den)
    rst = asum[:, :128] / den + b2_r[...]

    @pl.when(p == 0)
    def _():
        @pl.when(i == 0)
        def _():
            stats_r[...] = jnp.zeros_like(stats_r)

        stats_r[0:1] += jnp.sum(rst, axis=0, keepdims=True)
        stats_r[1:2] += jnp.sum(rst * rst, axis=0, keepdims=True)

    @pl.when(p == 1)
    def _():
        n_total = pl.num_programs(1) * rst.shape[0]
        mu = stats_r[0:1] / n_total
        var = stats_r[1:2] / n_total - mu * mu
        hbn = (rst - mu) * lax.rsqrt(var + 1e-5) * g2_r[...] + be2_r[...]
        h3 = h2_r[...] + _elu(hbn)

        @pl.when(i == 0)
        def _():
            hsum_r[...] = jnp.zeros_like(hsum_r)

        hsum_r[...] += jnp.sum(h3, axis=0, keepdims=True)

        @pl.when(i == pl.num_programs(1) - 1)
        def _():
            hg = hsum_r[...] / n_total
            y = jnp.maximum(jnp.dot(hg, wr0_r[...], preferred_element_type=jnp.float32)
                            + br0_r[...], 0.0)
            y = jnp.maximum(jnp.dot(y, wr1_r[...], preferred_element_type=jnp.float32)
                            + br1_r[...], 0.0)
            y = jnp.dot(y, wr2_r[...], preferred_element_type=jnp.float32) + br2_r[...]
            y_r[...] = jax.nn.sigmoid(y)


def _stage_c(acc, h2, b2, g2, be2, wr0, br0, wr1, br1, wr2, br2):
    n = h2.shape[0]
    grid = n // _N_BLK
    return pl.pallas_call(
        _body_c,
        grid=(2, grid),
        in_specs=[
            pl.BlockSpec((2, _N_BLK, 144), lambda p, i: (0, i, 0)),
            pl.BlockSpec((_N_BLK, 128), lambda p, i: (i, 0)),
            pl.BlockSpec((1, 128), lambda p, i: (0, 0)),
            pl.BlockSpec((1, 128), lambda p, i: (0, 0)),
            pl.BlockSpec((1, 128), lambda p, i: (0, 0)),
            pl.BlockSpec((128, 64), lambda p, i: (0, 0)),
            pl.BlockSpec((1, 64), lambda p, i: (0, 0)),
            pl.BlockSpec((64, 32), lambda p, i: (0, 0)),
            pl.BlockSpec((1, 32), lambda p, i: (0, 0)),
            pl.BlockSpec((32, 128), lambda p, i: (0, 0)),
            pl.BlockSpec((1, 128), lambda p, i: (0, 0)),
        ],
        out_specs=pl.BlockSpec((1, 128), lambda p, i: (0, 0)),
        out_shape=jax.ShapeDtypeStruct((1, 128), jnp.float32),
        scratch_shapes=[pltpu.VMEM((2, 128), jnp.float32),
                        pltpu.VMEM((1, 128), jnp.float32)],
    )(acc, h2, b2, g2, be2, wr0, br0, wr1, br1, wr2, br2)


_C = 128
_NW = 32
_DW = 144


def _edge_body(lane_map, n_pad, e_total,
               tbl_hbm, er_hbm, b_hbm, edge_hbm, out_hbm,
               idx_v, rows_v, err_v, b_v, acc_sh, sem_a, sem_b):
    import jax.experimental.pallas.tpu_sc as plsc

    cid = lax.axis_index("c")
    sid = lax.axis_index("s")
    wid = sid * 2 + cid
    nchunks = e_total // _C
    rows_per_tile = n_pad // 16
    zvec = jnp.zeros((16,), jnp.float32)
    iota16 = lax.iota(jnp.int32, 16)

    def full16(v):
        return jnp.full((16,), v, jnp.int32)

    def _zrow(i, _):
        for j in range(_DW // 16):
            rows_v[i, pl.ds(16 * j, 16)] = zvec
        return 0

    lax.fori_loop(0, _C, _zrow, 0)
    for t in range(rows_per_tile // _C):
        pltpu.sync_copy(rows_v,
                        acc_sh.at[pl.ds(sid * rows_per_tile + t * _C, _C), :])
    plsc.subcore_barrier()

    pltpu.sync_copy(b_hbm, b_v)
    bh = [b_v[h, :] for h in range(8)]

    nk = (nchunks - wid + _NW - 1) // _NW

    def _chunk(k, _):
        off = (wid + k * _NW) * _C
        pltpu.sync_copy(edge_hbm.at[:, pl.ds(off, _C)], idx_v)
        ga = pltpu.async_copy(tbl_hbm.at[idx_v.at[0]], rows_v, sem_a)
        gb = pltpu.async_copy(er_hbm.at[idx_v.at[1]], err_v, sem_b)
        ga.wait()
        gb.wait()

        for g in range(_C // 16):
            eidx = iota16 + (16 * g)
            for h in range(8):
                el = plsc.load_gather(rows_v, [eidx, full16(128 + h)])
                er = plsc.load_gather(err_v, [eidx, full16(h)])
                t = el + er
                t = jnp.maximum(t, 0.2 * t) - bh[h]
                plsc.store_scatter(rows_v, [eidx, full16(128 + h)], jnp.exp(t))

        def _edge(e, _):
            ef = full16(e)
            wbs = {lm: plsc.load_gather(rows_v, [ef, full16(128 + lm)])
                   for lm in sorted(set(lane_map))}
            for j in range(8):
                f = rows_v[e, pl.ds(16 * j, 16)]
                rows_v[e, pl.ds(16 * j, 16)] = f * wbs[lane_map[j]]
            return 0

        lax.fori_loop(0, _C, _edge, 0)
        pltpu.sync_copy(rows_v, acc_sh.at[idx_v.at[1]], add=True)
        return 0

    lax.fori_loop(0, nk, _chunk, 0)
    plsc.subcore_barrier()

    for t in range(rows_per_tile // _C):
        r0 = sid * rows_per_tile + t * _C
        pltpu.sync_copy(acc_sh.at[pl.ds(r0, _C), :], rows_v)
        pltpu.sync_copy(rows_v, out_hbm.at[cid, pl.ds(r0, _C), :])


def _edge_phase_sc(tbl, er16, b16, edge_index, heads):
    import jax.experimental.pallas.tpu_sc as plsc

    n = tbl.shape[0]
    n_pad = ((n + 2047) // 2048) * 2048
    e_total = edge_index.shape[1]
    lane_map = tuple(j if heads == 8 else 0 for j in range(8))
    mesh = plsc.VectorSubcoreMesh(core_axis_name="c", subcore_axis_name="s",
                                  num_cores=2, num_subcores=16)
    body = functools.partial(_edge_body, lane_map, n_pad, e_total)
    return pl.kernel(
        body,
        out_type=jax.ShapeDtypeStruct((2, n_pad, _DW), jnp.float32),
        mesh=mesh,
        compiler_params=pltpu.CompilerParams(use_tc_tiling_on_sc=False,
                                             needs_layout_passes=False),
        scratch_types=[
            pltpu.VMEM((2, _C), jnp.int32),
            pltpu.VMEM((_C, _DW), jnp.float32),
            pltpu.VMEM((_C, 16), jnp.float32),
            pltpu.VMEM((16, 16), jnp.float32),
            pltpu.VMEM_SHARED((n_pad, _DW), jnp.float32),
            pltpu.SemaphoreType.DMA,
            pltpu.SemaphoreType.DMA,
        ],
    )(tbl, er16, jnp.broadcast_to(b16[:, None], (16, 16)), edge_index)


def _edge_phase_jnp(tbl, er16, b16, src, dst, heads):
    n = tbl.shape[0]
    el = tbl[:, 128:144]
    w = jnp.exp(_lrelu(el[src] + er16[dst]) - b16[None, :])
    feat = tbl[:, :128]
    wexp = jnp.repeat(w[:, :heads], 128 // heads, axis=1)
    v = jnp.concatenate([feat[src] * wexp, w], axis=1)
    acc = jax.ops.segment_sum(v, dst, num_segments=n)
    return jnp.stack([acc, jnp.zeros_like(acc)])


def kernel(x, edge_index, W_in, b_in, W1, al1, ar1, b1, g1, be1,
           W2, al2, ar2, b2, g2, be2, Wr0, br0, Wr1, br1, Wr2, br2):
    heads, hid = al1.shape
    gat_out = al2.shape[1]

    eye = jnp.eye(heads, dtype=jnp.float32)
    al1_blk = (al1[:, :, None] * eye[:, None, :]).reshape(heads * hid, heads)
    al1_blk = jnp.pad(al1_blk, ((0, 0), (0, 16 - heads)))
    ar1_blk = (ar1[:, :, None] * eye[:, None, :]).reshape(heads * hid, heads)
    ar1_blk = jnp.pad(ar1_blk, ((0, 0), (0, 16 - heads)))
    al2_blk = jnp.pad(al2.T, ((0, 0), (0, 15)))
    ar2_blk = jnp.pad(ar2.T, ((0, 0), (0, 15)))
    denp = jnp.repeat(eye, hid, axis=1)

    row = lambda v: v.reshape(1, -1)
    src = edge_index[0]
    dst = edge_index[1]

    h, tbl1, er1, mel1, mer1 = _stage_a(x, W_in, row(b_in), W1, al1_blk, ar1_blk)
    b1v = _lrelu(mel1 + mer1)[0]
    acc1 = _edge_phase_sc(tbl1, er1, b1v, edge_index, heads)
    h2, tbl2, er2, mel2, mer2 = _stage_b(acc1, h, denp, row(b1), row(g1), row(be1),
                                         W2, al2_blk, ar2_blk)
    b2v = _lrelu(mel2 + mer2)[0]
    acc2 = _edge_phase_sc(tbl2, er2, b2v, edge_index, 1)
    return _stage_c(acc2, h2, row(b2), row(g2), row(be2),
                    Wr0, row(br0), Wr1, row(br1), Wr2, row(br2))


# speedup vs baseline: 50.7230x; 1.0158x over previous
import functools

import jax
import jax.numpy as jnp
from jax import lax
from jax.experimental import pallas as pl
from jax.experimental.pallas import tpu as pltpu

_N_BLK = 2000


def _elu(x):
    return jnp.where(x > 0, x, jnp.exp(x) - 1.0)


def _lrelu(x):
    return jnp.maximum(x, 0.2 * x)


def _body_a(x_r, win_r, bin_r, w1_r, al_r, ar_r,
            h_r, tbl_r, er_r, mel_r, mer_r):
    i = pl.program_id(0)
    h = jnp.dot(x_r[...], win_r[...], preferred_element_type=jnp.float32) + bin_r[...]
    feat = jnp.dot(h, w1_r[...], preferred_element_type=jnp.float32)
    el = jnp.dot(feat, al_r[...], preferred_element_type=jnp.float32)
    er = jnp.dot(feat, ar_r[...], preferred_element_type=jnp.float32)
    h_r[...] = h
    tbl_r[:, :128] = feat
    tbl_r[:, 128:144] = el
    er_r[...] = er

    @pl.when(i == 0)
    def _():
        mel_r[...] = jnp.full((1, 16), -1e30, jnp.float32)
        mer_r[...] = jnp.full((1, 16), -1e30, jnp.float32)

    mel_r[...] = jnp.maximum(mel_r[...], jnp.max(el, axis=0, keepdims=True))
    mer_r[...] = jnp.maximum(mer_r[...], jnp.max(er, axis=0, keepdims=True))


def _stage_a(x, w_in, b_in, w1, al_blk, ar_blk):
    n = x.shape[0]
    grid = n // _N_BLK
    return pl.pallas_call(
        _body_a,
        grid=(grid,),
        in_specs=[
            pl.BlockSpec((_N_BLK, 128), lambda i: (i, 0)),
            pl.BlockSpec((128, 128), lambda i: (0, 0)),
            pl.BlockSpec((1, 128), lambda i: (0, 0)),
            pl.BlockSpec((128, 128), lambda i: (0, 0)),
            pl.BlockSpec((128, 16), lambda i: (0, 0)),
            pl.BlockSpec((128, 16), lambda i: (0, 0)),
        ],
        out_specs=[
            pl.BlockSpec((_N_BLK, 128), lambda i: (i, 0)),
            pl.BlockSpec((_N_BLK, 144), lambda i: (i, 0)),
            pl.BlockSpec((_N_BLK, 16), lambda i: (i, 0)),
            pl.BlockSpec((1, 16), lambda i: (0, 0)),
            pl.BlockSpec((1, 16), lambda i: (0, 0)),
        ],
        out_shape=[
            jax.ShapeDtypeStruct((n, 128), jnp.float32),
            jax.ShapeDtypeStruct((n, 144), jnp.float32),
            jax.ShapeDtypeStruct((n, 16), jnp.float32),
            jax.ShapeDtypeStruct((1, 16), jnp.float32),
            jax.ShapeDtypeStruct((1, 16), jnp.float32),
        ],
    )(x, w_in, b_in, w1, al_blk, ar_blk)


def _body_b(acc_r, h_r, denp_r, b1_r, g1_r, be1_r, w2_r, al2_r, ar2_r,
            h2_r, tbl2_r, er2_r, mel2_r, mer2_r, stats_r):
    p = pl.program_id(0)
    i = pl.program_id(1)
    asum = acc_r[0] + acc_r[1]
    den = jnp.dot(asum[:, 128:136], denp_r[...],
                  preferred_element_type=jnp.float32)
    den = jnp.where(den == 0.0, 1.0, den)
    rst = asum[:, :128] / den + b1_r[...]

    @pl.when(p == 0)
    def _():
        @pl.when(i == 0)
        def _():
            stats_r[...] = jnp.zeros_like(stats_r)

        stats_r[0:1] += jnp.sum(rst, axis=0, keepdims=True)
        stats_r[1:2] += jnp.sum(rst * rst, axis=0, keepdims=True)

    @pl.when(p == 1)
    def _():
        n_total = pl.num_programs(1) * rst.shape[0]
        mu = stats_r[0:1] / n_total
        var = stats_r[1:2] / n_total - mu * mu
        hbn = (rst - mu) * lax.rsqrt(var + 1e-5) * g1_r[...] + be1_r[...]
        h2 = h_r[...] + _elu(hbn)
        h2_r[...] = h2
        feat2 = jnp.dot(h2, w2_r[...], preferred_element_type=jnp.float32)
        el2 = jnp.dot(feat2, al2_r[...], preferred_element_type=jnp.float32)
        er2 = jnp.dot(feat2, ar2_r[...], preferred_element_type=jnp.float32)
        tbl2_r[:, :128] = feat2
        tbl2_r[:, 128:144] = el2
        er2_r[...] = er2

        @pl.when(i == 0)
        def _():
            mel2_r[...] = jnp.full((1, 16), -1e30, jnp.float32)
            mer2_r[...] = jnp.full((1, 16), -1e30, jnp.float32)

        mel2_r[...] = jnp.maximum(mel2_r[...], jnp.max(el2, axis=0, keepdims=True))
        mer2_r[...] = jnp.maximum(mer2_r[...], jnp.max(er2, axis=0, keepdims=True))


def _stage_b(acc, h, denp, b1, g1, be1, w2, al2_blk, ar2_blk):
    n = h.shape[0]
    grid = n // _N_BLK
    return pl.pallas_call(
        _body_b,
        grid=(2, grid),
        in_specs=[
            pl.BlockSpec((2, _N_BLK, 144), lambda p, i: (0, i, 0)),
            pl.BlockSpec((_N_BLK, 128), lambda p, i: (i, 0)),
            pl.BlockSpec((8, 128), lambda p, i: (0, 0)),
            pl.BlockSpec((1, 128), lambda p, i: (0, 0)),
            pl.BlockSpec((1, 128), lambda p, i: (0, 0)),
            pl.BlockSpec((1, 128), lambda p, i: (0, 0)),
            pl.BlockSpec((128, 128), lambda p, i: (0, 0)),
            pl.BlockSpec((128, 16), lambda p, i: (0, 0)),
            pl.BlockSpec((128, 16), lambda p, i: (0, 0)),
        ],
        out_specs=[
            pl.BlockSpec((_N_BLK, 128), lambda p, i: (i, 0)),
            pl.BlockSpec((_N_BLK, 144), lambda p, i: (i, 0)),
            pl.BlockSpec((_N_BLK, 16), lambda p, i: (i, 0)),
            pl.BlockSpec((1, 16), lambda p, i: (0, 0)),
            pl.BlockSpec((1, 16), lambda p, i: (0, 0)),
        ],
        out_shape=[
            jax.ShapeDtypeStruct((n, 128), jnp.float32),
            jax.ShapeDtypeStruct((n, 144), jnp.float32),
            jax.ShapeDtypeStruct((n, 16), jnp.float32),
            jax.ShapeDtypeStruct((1, 16), jnp.float32),
            jax.ShapeDtypeStruct((1, 16), jnp.float32),
        ],
        scratch_shapes=[pltpu.VMEM((2, 128), jnp.float32)],
    )(acc, h, denp, b1, g1, be1, w2, al2_blk, ar2_blk)


def _body_c(acc_r, h2_r, b2_r, g2_r, be2_r,
            wr0_r, br0_r, wr1_r, br1_r, wr2_r, br2_r,
            y_r, stats_r, hsum_r):
    p = pl.program_id(0)
    i = pl.program_id(1)
    asum = acc_r[0] + acc_r[1]
    den = lax.broadcast_in_dim(asum[:, 128:129], asum[:, :128].shape, (0, 1))
    den = jnp.where(den == 0.0, 1.0, den)
    rst = asum[:, :128] / den + b2_r[...]

    @pl.when(p == 0)
    def _():
        @pl.when(i == 0)
        def _():
            stats_r[...] = jnp.zeros_like(stats_r)

        stats_r[0:1] += jnp.sum(rst, axis=0, keepdims=True)
        stats_r[1:2] += jnp.sum(rst * rst, axis=0, keepdims=True)

    @pl.when(p == 1)
    def _():
        n_total = pl.num_programs(1) * rst.shape[0]
        mu = stats_r[0:1] / n_total
        var = stats_r[1:2] / n_total - mu * mu
        hbn = (rst - mu) * lax.rsqrt(var + 1e-5) * g2_r[...] + be2_r[...]
        h3 = h2_r[...] + _elu(hbn)

        @pl.when(i == 0)
        def _():
            hsum_r[...] = jnp.zeros_like(hsum_r)

        hsum_r[...] += jnp.sum(h3, axis=0, keepdims=True)

        @pl.when(i == pl.num_programs(1) - 1)
        def _():
            hg = hsum_r[...] / n_total
            y = jnp.maximum(jnp.dot(hg, wr0_r[...], preferred_element_type=jnp.float32)
                            + br0_r[...], 0.0)
            y = jnp.maximum(jnp.dot(y, wr1_r[...], preferred_element_type=jnp.float32)
                            + br1_r[...], 0.0)
            y = jnp.dot(y, wr2_r[...], preferred_element_type=jnp.float32) + br2_r[...]
            y_r[...] = jax.nn.sigmoid(y)


def _stage_c(acc, h2, b2, g2, be2, wr0, br0, wr1, br1, wr2, br2):
    n = h2.shape[0]
    grid = n // _N_BLK
    return pl.pallas_call(
        _body_c,
        grid=(2, grid),
        in_specs=[
            pl.BlockSpec((2, _N_BLK, 144), lambda p, i: (0, i, 0)),
            pl.BlockSpec((_N_BLK, 128), lambda p, i: (i, 0)),
            pl.BlockSpec((1, 128), lambda p, i: (0, 0)),
            pl.BlockSpec((1, 128), lambda p, i: (0, 0)),
            pl.BlockSpec((1, 128), lambda p, i: (0, 0)),
            pl.BlockSpec((128, 64), lambda p, i: (0, 0)),
            pl.BlockSpec((1, 64), lambda p, i: (0, 0)),
            pl.BlockSpec((64, 32), lambda p, i: (0, 0)),
            pl.BlockSpec((1, 32), lambda p, i: (0, 0)),
            pl.BlockSpec((32, 128), lambda p, i: (0, 0)),
            pl.BlockSpec((1, 128), lambda p, i: (0, 0)),
        ],
        out_specs=pl.BlockSpec((1, 128), lambda p, i: (0, 0)),
        out_shape=jax.ShapeDtypeStruct((1, 128), jnp.float32),
        scratch_shapes=[pltpu.VMEM((2, 128), jnp.float32),
                        pltpu.VMEM((1, 128), jnp.float32)],
    )(acc, h2, b2, g2, be2, wr0, br0, wr1, br1, wr2, br2)


_C = 128
_NW = 32
_DW = 144


def _edge_body(lane_map, n_pad, e_total,
               tbl_hbm, er_hbm, b_hbm, edge_hbm, out_hbm,
               idx_v, rows_v, err_v, b_v, acc_sh, sem_a, sem_b):
    import jax.experimental.pallas.tpu_sc as plsc

    cid = lax.axis_index("c")
    sid = lax.axis_index("s")
    wid = sid * 2 + cid
    nchunks = e_total // _C
    rows_per_tile = n_pad // 16
    zvec = jnp.zeros((16,), jnp.float32)

    def _zrow(i, _):
        for j in range(_DW // 16):
            rows_v[i, pl.ds(16 * j, 16)] = zvec
        return 0

    lax.fori_loop(0, _C, _zrow, 0)
    for t in range(rows_per_tile // _C):
        pltpu.sync_copy(rows_v,
                        acc_sh.at[pl.ds(sid * rows_per_tile + t * _C, _C), :])
    plsc.subcore_barrier()

    pltpu.sync_copy(b_hbm, b_v)
    bvec = b_v[0, :]
    bidx = [jnp.full((16, 1), lm, jnp.int32) for lm in range(8)]
    gdn = lax.GatherDimensionNumbers(offset_dims=(), collapsed_slice_dims=(0,),
                                     start_index_map=(0,))

    def _bcast(w, lm):
        return lax.gather(w, bidx[lm], gdn, (1,),
                          mode=lax.GatherScatterMode.PROMISE_IN_BOUNDS)

    nk = (nchunks - wid + _NW - 1) // _NW

    def _chunk(k, _):
        off = (wid + k * _NW) * _C
        pltpu.sync_copy(edge_hbm.at[:, pl.ds(off, _C)], idx_v)
        ga = pltpu.async_copy(tbl_hbm.at[idx_v.at[0]], rows_v, sem_a)
        gb = pltpu.async_copy(er_hbm.at[idx_v.at[1]], err_v, sem_b)
        ga.wait()
        gb.wait()

        def _edge(e, _):
            t = rows_v[e, pl.ds(128, 16)] + err_v[e, :]
            w = jnp.exp(jnp.maximum(t, 0.2 * t) - bvec)
            rows_v[e, pl.ds(128, 16)] = w
            wbs = {lm: _bcast(w, lm) for lm in sorted(set(lane_map))}
            for j in range(8):
                f = rows_v[e, pl.ds(16 * j, 16)]
                rows_v[e, pl.ds(16 * j, 16)] = f * wbs[lane_map[j]]
            return 0

        lax.fori_loop(0, _C, _edge, 0)
        pltpu.sync_copy(rows_v, acc_sh.at[idx_v.at[1]], add=True)
        return 0

    lax.fori_loop(0, nk, _chunk, 0)
    plsc.subcore_barrier()

    for t in range(rows_per_tile // _C):
        r0 = sid * rows_per_tile + t * _C
        pltpu.sync_copy(acc_sh.at[pl.ds(r0, _C), :], rows_v)
        pltpu.sync_copy(rows_v, out_hbm.at[cid, pl.ds(r0, _C), :])


def _edge_phase_sc(tbl, er16, b16, edge_index, heads):
    import jax.experimental.pallas.tpu_sc as plsc

    n = tbl.shape[0]
    n_pad = ((n + 2047) // 2048) * 2048
    e_total = edge_index.shape[1]
    lane_map = tuple(j if heads == 8 else 0 for j in range(8))
    mesh = plsc.VectorSubcoreMesh(core_axis_name="c", subcore_axis_name="s",
                                  num_cores=2, num_subcores=16)
    body = functools.partial(_edge_body, lane_map, n_pad, e_total)
    return pl.kernel(
        body,
        out_type=jax.ShapeDtypeStruct((2, n_pad, _DW), jnp.float32),
        mesh=mesh,
        compiler_params=pltpu.CompilerParams(use_tc_tiling_on_sc=False,
                                             needs_layout_passes=False),
        scratch_types=[
            pltpu.VMEM((2, _C), jnp.int32),
            pltpu.VMEM((_C, _DW), jnp.float32),
            pltpu.VMEM((_C, 16), jnp.float32),
            pltpu.VMEM((1, 16), jnp.float32),
            pltpu.VMEM_SHARED((n_pad, _DW), jnp.float32),
            pltpu.SemaphoreType.DMA,
            pltpu.SemaphoreType.DMA,
        ],
    )(tbl, er16, b16.reshape(1, 16), edge_index)


def _edge_phase_jnp(tbl, er16, b16, src, dst, heads):
    n = tbl.shape[0]
    el = tbl[:, 128:144]
    w = jnp.exp(_lrelu(el[src] + er16[dst]) - b16[None, :])
    feat = tbl[:, :128]
    wexp = jnp.repeat(w[:, :heads], 128 // heads, axis=1)
    v = jnp.concatenate([feat[src] * wexp, w], axis=1)
    acc = jax.ops.segment_sum(v, dst, num_segments=n)
    return jnp.stack([acc, jnp.zeros_like(acc)])


def kernel(x, edge_index, W_in, b_in, W1, al1, ar1, b1, g1, be1,
           W2, al2, ar2, b2, g2, be2, Wr0, br0, Wr1, br1, Wr2, br2):
    heads, hid = al1.shape
    gat_out = al2.shape[1]

    eye = jnp.eye(heads, dtype=jnp.float32)
    al1_blk = (al1[:, :, None] * eye[:, None, :]).reshape(heads * hid, heads)
    al1_blk = jnp.pad(al1_blk, ((0, 0), (0, 16 - heads)))
    ar1_blk = (ar1[:, :, None] * eye[:, None, :]).reshape(heads * hid, heads)
    ar1_blk = jnp.pad(ar1_blk, ((0, 0), (0, 16 - heads)))
    al2_blk = jnp.pad(al2.T, ((0, 0), (0, 15)))
    ar2_blk = jnp.pad(ar2.T, ((0, 0), (0, 15)))
    denp = jnp.repeat(eye, hid, axis=1)

    row = lambda v: v.reshape(1, -1)
    src = edge_index[0]
    dst = edge_index[1]

    h, tbl1, er1, mel1, mer1 = _stage_a(x, W_in, row(b_in), W1, al1_blk, ar1_blk)
    b1v = _lrelu(mel1 + mer1)[0]
    acc1 = _edge_phase_sc(tbl1, er1, b1v, edge_index, heads)
    h2, tbl2, er2, mel2, mer2 = _stage_b(acc1, h, denp, row(b1), row(g1), row(be1),
                                         W2, al2_blk, ar2_blk)
    b2v = _lrelu(mel2 + mer2)[0]
    acc2 = _edge_phase_sc(tbl2, er2, b2v, edge_index, 1)
    return _stage_c(acc2, h2, row(b2), row(g2), row(be2),
                    Wr0, row(br0), Wr1, row(br1), Wr2, row(br2))


# speedup vs baseline: 71.7691x; 1.4149x over previous
import functools

import jax
import jax.numpy as jnp
from jax import lax
from jax.experimental import pallas as pl
from jax.experimental.pallas import tpu as pltpu

_N_BLK = 2000


def _elu(x):
    return jnp.where(x > 0, x, jnp.exp(x) - 1.0)


def _lrelu(x):
    return jnp.maximum(x, 0.2 * x)


def _body_a(x_r, win_r, bin_r, w1_r, al_r, ar_r,
            h_r, tbl_r, er_r, mel_r, mer_r):
    i = pl.program_id(0)
    h = jnp.dot(x_r[...], win_r[...], preferred_element_type=jnp.float32) + bin_r[...]
    feat = jnp.dot(h, w1_r[...], preferred_element_type=jnp.float32)
    el = jnp.dot(feat, al_r[...], preferred_element_type=jnp.float32)
    er = jnp.dot(feat, ar_r[...], preferred_element_type=jnp.float32)
    h_r[...] = h
    tbl_r[:, :128] = feat
    tbl_r[:, 128:144] = el
    er_r[...] = er

    @pl.when(i == 0)
    def _():
        mel_r[...] = jnp.full((1, 16), -1e30, jnp.float32)
        mer_r[...] = jnp.full((1, 16), -1e30, jnp.float32)

    mel_r[...] = jnp.maximum(mel_r[...], jnp.max(el, axis=0, keepdims=True))
    mer_r[...] = jnp.maximum(mer_r[...], jnp.max(er, axis=0, keepdims=True))


def _stage_a(x, w_in, b_in, w1, al_blk, ar_blk):
    n = x.shape[0]
    grid = n // _N_BLK
    return pl.pallas_call(
        _body_a,
        grid=(grid,),
        in_specs=[
            pl.BlockSpec((_N_BLK, 128), lambda i: (i, 0)),
            pl.BlockSpec((128, 128), lambda i: (0, 0)),
            pl.BlockSpec((1, 128), lambda i: (0, 0)),
            pl.BlockSpec((128, 128), lambda i: (0, 0)),
            pl.BlockSpec((128, 16), lambda i: (0, 0)),
            pl.BlockSpec((128, 16), lambda i: (0, 0)),
        ],
        out_specs=[
            pl.BlockSpec((_N_BLK, 128), lambda i: (i, 0)),
            pl.BlockSpec((_N_BLK, 144), lambda i: (i, 0)),
            pl.BlockSpec((_N_BLK, 16), lambda i: (i, 0)),
            pl.BlockSpec((1, 16), lambda i: (0, 0)),
            pl.BlockSpec((1, 16), lambda i: (0, 0)),
        ],
        out_shape=[
            jax.ShapeDtypeStruct((n, 128), jnp.float32),
            jax.ShapeDtypeStruct((n, 144), jnp.float32),
            jax.ShapeDtypeStruct((n, 16), jnp.float32),
            jax.ShapeDtypeStruct((1, 16), jnp.float32),
            jax.ShapeDtypeStruct((1, 16), jnp.float32),
        ],
    )(x, w_in, b_in, w1, al_blk, ar_blk)


def _body_b(acc_r, h_r, denp_r, b1_r, g1_r, be1_r, w2_r, al2_r, ar2_r,
            h2_r, tbl2_r, er2_r, mel2_r, mer2_r, stats_r):
    p = pl.program_id(0)
    i = pl.program_id(1)
    asum = acc_r[0] + acc_r[1]
    den = jnp.dot(asum[:, 128:136], denp_r[...],
                  preferred_element_type=jnp.float32)
    den = jnp.where(den == 0.0, 1.0, den)
    rst = asum[:, :128] / den + b1_r[...]

    @pl.when(p == 0)
    def _():
        @pl.when(i == 0)
        def _():
            stats_r[...] = jnp.zeros_like(stats_r)

        stats_r[0:1] += jnp.sum(rst, axis=0, keepdims=True)
        stats_r[1:2] += jnp.sum(rst * rst, axis=0, keepdims=True)

    @pl.when(p == 1)
    def _():
        n_total = pl.num_programs(1) * rst.shape[0]
        mu = stats_r[0:1] / n_total
        var = stats_r[1:2] / n_total - mu * mu
        hbn = (rst - mu) * lax.rsqrt(var + 1e-5) * g1_r[...] + be1_r[...]
        h2 = h_r[...] + _elu(hbn)
        h2_r[...] = h2
        feat2 = jnp.dot(h2, w2_r[...], preferred_element_type=jnp.float32)
        el2 = jnp.dot(feat2, al2_r[...], preferred_element_type=jnp.float32)
        er2 = jnp.dot(feat2, ar2_r[...], preferred_element_type=jnp.float32)
        tbl2_r[:, :128] = feat2
        tbl2_r[:, 128:144] = el2
        er2_r[...] = er2

        @pl.when(i == 0)
        def _():
            mel2_r[...] = jnp.full((1, 16), -1e30, jnp.float32)
            mer2_r[...] = jnp.full((1, 16), -1e30, jnp.float32)

        mel2_r[...] = jnp.maximum(mel2_r[...], jnp.max(el2, axis=0, keepdims=True))
        mer2_r[...] = jnp.maximum(mer2_r[...], jnp.max(er2, axis=0, keepdims=True))


def _stage_b(acc, h, denp, b1, g1, be1, w2, al2_blk, ar2_blk):
    n = h.shape[0]
    grid = n // _N_BLK
    return pl.pallas_call(
        _body_b,
        grid=(2, grid),
        in_specs=[
            pl.BlockSpec((2, _N_BLK, 144), lambda p, i: (0, i, 0)),
            pl.BlockSpec((_N_BLK, 128), lambda p, i: (i, 0)),
            pl.BlockSpec((8, 128), lambda p, i: (0, 0)),
            pl.BlockSpec((1, 128), lambda p, i: (0, 0)),
            pl.BlockSpec((1, 128), lambda p, i: (0, 0)),
            pl.BlockSpec((1, 128), lambda p, i: (0, 0)),
            pl.BlockSpec((128, 128), lambda p, i: (0, 0)),
            pl.BlockSpec((128, 16), lambda p, i: (0, 0)),
            pl.BlockSpec((128, 16), lambda p, i: (0, 0)),
        ],
        out_specs=[
            pl.BlockSpec((_N_BLK, 128), lambda p, i: (i, 0)),
            pl.BlockSpec((_N_BLK, 144), lambda p, i: (i, 0)),
            pl.BlockSpec((_N_BLK, 16), lambda p, i: (i, 0)),
            pl.BlockSpec((1, 16), lambda p, i: (0, 0)),
            pl.BlockSpec((1, 16), lambda p, i: (0, 0)),
        ],
        out_shape=[
            jax.ShapeDtypeStruct((n, 128), jnp.float32),
            jax.ShapeDtypeStruct((n, 144), jnp.float32),
            jax.ShapeDtypeStruct((n, 16), jnp.float32),
            jax.ShapeDtypeStruct((1, 16), jnp.float32),
            jax.ShapeDtypeStruct((1, 16), jnp.float32),
        ],
        scratch_shapes=[pltpu.VMEM((2, 128), jnp.float32)],
    )(acc, h, denp, b1, g1, be1, w2, al2_blk, ar2_blk)


def _body_c(acc_r, h2_r, b2_r, g2_r, be2_r,
            wr0_r, br0_r, wr1_r, br1_r, wr2_r, br2_r,
            y_r, stats_r, hsum_r):
    p = pl.program_id(0)
    i = pl.program_id(1)
    asum = acc_r[0] + acc_r[1]
    den = lax.broadcast_in_dim(asum[:, 128:129], asum[:, :128].shape, (0, 1))
    den = jnp.where(den == 0.0, 1.0, den)
    rst = asum[:, :128] / den + b2_r[...]

    @pl.when(p == 0)
    def _():
        @pl.when(i == 0)
        def _():
            stats_r[...] = jnp.zeros_like(stats_r)

        stats_r[0:1] += jnp.sum(rst, axis=0, keepdims=True)
        stats_r[1:2] += jnp.sum(rst * rst, axis=0, keepdims=True)

    @pl.when(p == 1)
    def _():
        n_total = pl.num_programs(1) * rst.shape[0]
        mu = stats_r[0:1] / n_total
        var = stats_r[1:2] / n_total - mu * mu
        hbn = (rst - mu) * lax.rsqrt(var + 1e-5) * g2_r[...] + be2_r[...]
        h3 = h2_r[...] + _elu(hbn)

        @pl.when(i == 0)
        def _():
            hsum_r[...] = jnp.zeros_like(hsum_r)

        hsum_r[...] += jnp.sum(h3, axis=0, keepdims=True)

        @pl.when(i == pl.num_programs(1) - 1)
        def _():
            hg = hsum_r[...] / n_total
            y = jnp.maximum(jnp.dot(hg, wr0_r[...], preferred_element_type=jnp.float32)
                            + br0_r[...], 0.0)
            y = jnp.maximum(jnp.dot(y, wr1_r[...], preferred_element_type=jnp.float32)
                            + br1_r[...], 0.0)
            y = jnp.dot(y, wr2_r[...], preferred_element_type=jnp.float32) + br2_r[...]
            y_r[...] = jax.nn.sigmoid(y)


def _stage_c(acc, h2, b2, g2, be2, wr0, br0, wr1, br1, wr2, br2):
    n = h2.shape[0]
    grid = n // _N_BLK
    return pl.pallas_call(
        _body_c,
        grid=(2, grid),
        in_specs=[
            pl.BlockSpec((2, _N_BLK, 144), lambda p, i: (0, i, 0)),
            pl.BlockSpec((_N_BLK, 128), lambda p, i: (i, 0)),
            pl.BlockSpec((1, 128), lambda p, i: (0, 0)),
            pl.BlockSpec((1, 128), lambda p, i: (0, 0)),
            pl.BlockSpec((1, 128), lambda p, i: (0, 0)),
            pl.BlockSpec((128, 64), lambda p, i: (0, 0)),
            pl.BlockSpec((1, 64), lambda p, i: (0, 0)),
            pl.BlockSpec((64, 32), lambda p, i: (0, 0)),
            pl.BlockSpec((1, 32), lambda p, i: (0, 0)),
            pl.BlockSpec((32, 128), lambda p, i: (0, 0)),
            pl.BlockSpec((1, 128), lambda p, i: (0, 0)),
        ],
        out_specs=pl.BlockSpec((1, 128), lambda p, i: (0, 0)),
        out_shape=jax.ShapeDtypeStruct((1, 128), jnp.float32),
        scratch_shapes=[pltpu.VMEM((2, 128), jnp.float32),
                        pltpu.VMEM((1, 128), jnp.float32)],
    )(acc, h2, b2, g2, be2, wr0, br0, wr1, br1, wr2, br2)


_C = 80
_NW = 32
_DW = 144


def _edge_body(lane_map, n_pad, e_total,
               tbl_hbm, er_hbm, b_hbm, edge_hbm, out_hbm,
               idxs_v, idxd_v, rows0, rows1, err0, err1, b_v, acc_sh,
               sg0, sg1, ss0, ss1, si):
    import jax.experimental.pallas.tpu_sc as plsc

    cid = lax.axis_index("c")
    sid = lax.axis_index("s")
    wid = sid * 2 + cid
    nk = e_total // _C // _NW
    rows_per_tile = n_pad // 16
    zvec = jnp.zeros((16,), jnp.float32)

    def _zrow(i, _):
        for j in range(_DW // 16):
            rows0[i, pl.ds(16 * j, 16)] = zvec
        return 0

    lax.fori_loop(0, _C, _zrow, 0)
    for t in range(rows_per_tile // _C):
        pltpu.sync_copy(rows0,
                        acc_sh.at[pl.ds(sid * rows_per_tile + t * _C, _C), :])
    plsc.subcore_barrier()

    pltpu.sync_copy(b_hbm, b_v)
    bvec = b_v[0, :]
    bidx = [jnp.full((16, 1), lm, jnp.int32) for lm in range(8)]
    gdn = lax.GatherDimensionNumbers(offset_dims=(), collapsed_slice_dims=(0,),
                                     start_index_map=(0,))

    def _bcast(w, lm):
        return lax.gather(w, bidx[lm], gdn, (1,),
                          mode=lax.GatherScatterMode.PROMISE_IN_BOUNDS)

    bufs = ((rows0, err0, sg0, ss0), (rows1, err1, sg1, ss1))

    def _off(k):
        return (wid + k * _NW) * _C

    def _idx_issue(k, m):
        pltpu.async_copy(edge_hbm.at[0, pl.ds(_off(k), _C)], idxs_v.at[m], si)
        pltpu.async_copy(edge_hbm.at[1, pl.ds(_off(k), _C)], idxd_v.at[m], si)

    def _idx_wait(k, m):
        pltpu.make_async_copy(edge_hbm.at[0, pl.ds(_off(k), _C)],
                              idxs_v.at[m], si).wait()
        pltpu.make_async_copy(edge_hbm.at[1, pl.ds(_off(k), _C)],
                              idxd_v.at[m], si).wait()

    def _gather_issue(m, rows, err, sg):
        pltpu.async_copy(tbl_hbm.at[idxs_v.at[m]], rows, sg)
        pltpu.async_copy(er_hbm.at[idxd_v.at[m]], err, sg)

    def _gather_wait(m, rows, err, sg):
        pltpu.make_async_copy(tbl_hbm.at[idxs_v.at[m]], rows, sg).wait()
        pltpu.make_async_copy(er_hbm.at[idxd_v.at[m]], err, sg).wait()

    def _scatter_issue(m, rows, ss):
        pltpu.async_copy(rows, acc_sh.at[idxd_v.at[m]], ss, add=True)

    def _scatter_wait(m, rows, ss):
        pltpu.make_async_copy(rows, acc_sh.at[idxd_v.at[m]], ss).wait()

    def _compute(rows, err):
        def _edge(e, _):
            t = rows[e, pl.ds(128, 16)] + err[e, :]
            w = jnp.exp(jnp.maximum(t, 0.2 * t) - bvec)
            rows[e, pl.ds(128, 16)] = w
            wbs = {lm: _bcast(w, lm) for lm in sorted(set(lane_map))}
            for j in range(8):
                f = rows[e, pl.ds(16 * j, 16)]
                rows[e, pl.ds(16 * j, 16)] = f * wbs[lane_map[j]]
            return 0

        lax.fori_loop(0, _C, _edge, 0)

    pltpu.sync_copy(edge_hbm.at[0, pl.ds(_off(0), _C)], idxs_v.at[0])
    pltpu.sync_copy(edge_hbm.at[1, pl.ds(_off(0), _C)], idxd_v.at[0])
    _gather_issue(0, rows0, err0, sg0)
    _idx_issue(1, 1)

    def _iter(k, _):
        m = lax.bitwise_and(k, 3)
        m1 = lax.bitwise_and(k + 1, 3)
        m2 = lax.bitwise_and(k + 2, 3)
        for p in (0, 1):
            @pl.when(lax.bitwise_and(k, 1) == p)
            def _():
                cur_rows, cur_err, sgp, ssp = bufs[p]
                nxt_rows, nxt_err, sgn, ssn = bufs[1 - p]

                @pl.when(k < nk - 1)
                def _():
                    _idx_wait(k + 1, m1)

                    @pl.when(k >= 1)
                    def _():
                        _scatter_wait(m1, nxt_rows, ssn)

                    _gather_issue(m1, nxt_rows, nxt_err, sgn)

                @pl.when(k < nk - 2)
                def _():
                    _idx_issue(k + 2, m2)

                _gather_wait(m, cur_rows, cur_err, sgp)
                _compute(cur_rows, cur_err)
                _scatter_issue(m, cur_rows, ssp)
        return 0

    lax.fori_loop(0, nk, _iter, 0)
    _scatter_wait(0, bufs[(nk - 2) & 1][0], bufs[(nk - 2) & 1][3])
    _scatter_wait(0, bufs[(nk - 1) & 1][0], bufs[(nk - 1) & 1][3])
    plsc.subcore_barrier()

    for t in range(rows_per_tile // _C):
        r0 = sid * rows_per_tile + t * _C
        pltpu.sync_copy(acc_sh.at[pl.ds(r0, _C), :], rows0)
        pltpu.sync_copy(rows0, out_hbm.at[cid, pl.ds(r0, _C), :])


def _edge_phase_sc(tbl, er16, b16, edge_index, heads):
    import jax.experimental.pallas.tpu_sc as plsc

    n = tbl.shape[0]
    n_pad = ((n + 2047) // 2048) * 2048
    e_total = edge_index.shape[1]
    lane_map = tuple(j if heads == 8 else 0 for j in range(8))
    mesh = plsc.VectorSubcoreMesh(core_axis_name="c", subcore_axis_name="s",
                                  num_cores=2, num_subcores=16)
    body = functools.partial(_edge_body, lane_map, n_pad, e_total)
    return pl.kernel(
        body,
        out_type=jax.ShapeDtypeStruct((2, n_pad, _DW), jnp.float32),
        mesh=mesh,
        compiler_params=pltpu.CompilerParams(use_tc_tiling_on_sc=False,
                                             needs_layout_passes=False),
        scratch_types=[
            pltpu.VMEM((4, _C), jnp.int32),
            pltpu.VMEM((4, _C), jnp.int32),
            pltpu.VMEM((_C, _DW), jnp.float32),
            pltpu.VMEM((_C, _DW), jnp.float32),
            pltpu.VMEM((_C, 16), jnp.float32),
            pltpu.VMEM((_C, 16), jnp.float32),
            pltpu.VMEM((1, 16), jnp.float32),
            pltpu.VMEM_SHARED((n_pad, _DW), jnp.float32),
            pltpu.SemaphoreType.DMA,
            pltpu.SemaphoreType.DMA,
            pltpu.SemaphoreType.DMA,
            pltpu.SemaphoreType.DMA,
            pltpu.SemaphoreType.DMA,
        ],
    )(tbl, er16, b16.reshape(1, 16), edge_index)


def _edge_phase_jnp(tbl, er16, b16, src, dst, heads):
    n = tbl.shape[0]
    el = tbl[:, 128:144]
    w = jnp.exp(_lrelu(el[src] + er16[dst]) - b16[None, :])
    feat = tbl[:, :128]
    wexp = jnp.repeat(w[:, :heads], 128 // heads, axis=1)
    v = jnp.concatenate([feat[src] * wexp, w], axis=1)
    acc = jax.ops.segment_sum(v, dst, num_segments=n)
    return jnp.stack([acc, jnp.zeros_like(acc)])


def kernel(x, edge_index, W_in, b_in, W1, al1, ar1, b1, g1, be1,
           W2, al2, ar2, b2, g2, be2, Wr0, br0, Wr1, br1, Wr2, br2):
    heads, hid = al1.shape
    gat_out = al2.shape[1]

    eye = jnp.eye(heads, dtype=jnp.float32)
    al1_blk = (al1[:, :, None] * eye[:, None, :]).reshape(heads * hid, heads)
    al1_blk = jnp.pad(al1_blk, ((0, 0), (0, 16 - heads)))
    ar1_blk = (ar1[:, :, None] * eye[:, None, :]).reshape(heads * hid, heads)
    ar1_blk = jnp.pad(ar1_blk, ((0, 0), (0, 16 - heads)))
    al2_blk = jnp.pad(al2.T, ((0, 0), (0, 15)))
    ar2_blk = jnp.pad(ar2.T, ((0, 0), (0, 15)))
    denp = jnp.repeat(eye, hid, axis=1)

    row = lambda v: v.reshape(1, -1)
    src = edge_index[0]
    dst = edge_index[1]

    h, tbl1, er1, mel1, mer1 = _stage_a(x, W_in, row(b_in), W1, al1_blk, ar1_blk)
    b1v = _lrelu(mel1 + mer1)[0]
    acc1 = _edge_phase_sc(tbl1, er1, b1v, edge_index, heads)
    h2, tbl2, er2, mel2, mer2 = _stage_b(acc1, h, denp, row(b1), row(g1), row(be1),
                                         W2, al2_blk, ar2_blk)
    b2v = _lrelu(mel2 + mer2)[0]
    acc2 = _edge_phase_sc(tbl2, er2, b2v, edge_index, 1)
    return _stage_c(acc2, h2, row(b2), row(g2), row(be2),
                    Wr0, row(br0), Wr1, row(br1), Wr2, row(br2))


# speedup vs baseline: 90.5371x; 1.2615x over previous
import functools

import jax
import jax.numpy as jnp
from jax import lax
from jax.experimental import pallas as pl
from jax.experimental.pallas import tpu as pltpu

_N_BLK = 2000


def _elu(x):
    return jnp.where(x > 0, x, jnp.exp(x) - 1.0)


def _lrelu(x):
    return jnp.maximum(x, 0.2 * x)


def _body_a(x_r, win_r, bin_r, w1_r, al_r, ar_r,
            h_r, tbl_r, er_r, mel_r, mer_r):
    i = pl.program_id(0)
    h = jnp.dot(x_r[...], win_r[...], preferred_element_type=jnp.float32) + bin_r[...]
    feat = jnp.dot(h, w1_r[...], preferred_element_type=jnp.float32)
    el = jnp.dot(feat, al_r[...], preferred_element_type=jnp.float32)
    er = jnp.dot(feat, ar_r[...], preferred_element_type=jnp.float32)
    h_r[...] = h
    tbl_r[:, :128] = feat
    tbl_r[:, 128:144] = el
    er_r[...] = er

    @pl.when(i == 0)
    def _():
        mel_r[...] = jnp.full((1, 16), -1e30, jnp.float32)
        mer_r[...] = jnp.full((1, 16), -1e30, jnp.float32)

    mel_r[...] = jnp.maximum(mel_r[...], jnp.max(el, axis=0, keepdims=True))
    mer_r[...] = jnp.maximum(mer_r[...], jnp.max(er, axis=0, keepdims=True))


def _stage_a(x, w_in, b_in, w1, al_blk, ar_blk):
    n = x.shape[0]
    grid = n // _N_BLK
    return pl.pallas_call(
        _body_a,
        grid=(grid,),
        in_specs=[
            pl.BlockSpec((_N_BLK, 128), lambda i: (i, 0)),
            pl.BlockSpec((128, 128), lambda i: (0, 0)),
            pl.BlockSpec((1, 128), lambda i: (0, 0)),
            pl.BlockSpec((128, 128), lambda i: (0, 0)),
            pl.BlockSpec((128, 16), lambda i: (0, 0)),
            pl.BlockSpec((128, 16), lambda i: (0, 0)),
        ],
        out_specs=[
            pl.BlockSpec((_N_BLK, 128), lambda i: (i, 0)),
            pl.BlockSpec((_N_BLK, 144), lambda i: (i, 0)),
            pl.BlockSpec((_N_BLK, 16), lambda i: (i, 0)),
            pl.BlockSpec((1, 16), lambda i: (0, 0)),
            pl.BlockSpec((1, 16), lambda i: (0, 0)),
        ],
        out_shape=[
            jax.ShapeDtypeStruct((n, 128), jnp.float32),
            jax.ShapeDtypeStruct((n, 144), jnp.float32),
            jax.ShapeDtypeStruct((n, 16), jnp.float32),
            jax.ShapeDtypeStruct((1, 16), jnp.float32),
            jax.ShapeDtypeStruct((1, 16), jnp.float32),
        ],
    )(x, w_in, b_in, w1, al_blk, ar_blk)


def _body_b(acc_r, h_r, denp_r, b1_r, g1_r, be1_r, w2_r, al2_r, ar2_r,
            h2_r, tbl2_r, er2_r, mel2_r, mer2_r, stats_r):
    p = pl.program_id(0)
    i = pl.program_id(1)
    asum = acc_r[0] + acc_r[1]
    den = jnp.dot(asum[:, 128:136], denp_r[...],
                  preferred_element_type=jnp.float32)
    den = jnp.where(den == 0.0, 1.0, den)
    rst = asum[:, :128] / den + b1_r[...]

    @pl.when(p == 0)
    def _():
        @pl.when(i == 0)
        def _():
            stats_r[...] = jnp.zeros_like(stats_r)

        stats_r[0:1] += jnp.sum(rst, axis=0, keepdims=True)
        stats_r[1:2] += jnp.sum(rst * rst, axis=0, keepdims=True)

    @pl.when(p == 1)
    def _():
        n_total = pl.num_programs(1) * rst.shape[0]
        mu = stats_r[0:1] / n_total
        var = stats_r[1:2] / n_total - mu * mu
        hbn = (rst - mu) * lax.rsqrt(var + 1e-5) * g1_r[...] + be1_r[...]
        h2 = h_r[...] + _elu(hbn)
        h2_r[...] = h2
        feat2 = jnp.dot(h2, w2_r[...], preferred_element_type=jnp.float32)
        el2 = jnp.dot(feat2, al2_r[...], preferred_element_type=jnp.float32)
        er2 = jnp.dot(feat2, ar2_r[...], preferred_element_type=jnp.float32)
        tbl2_r[:, :128] = feat2
        tbl2_r[:, 128:144] = el2
        er2_r[...] = er2

        @pl.when(i == 0)
        def _():
            mel2_r[...] = jnp.full((1, 16), -1e30, jnp.float32)
            mer2_r[...] = jnp.full((1, 16), -1e30, jnp.float32)

        mel2_r[...] = jnp.maximum(mel2_r[...], jnp.max(el2, axis=0, keepdims=True))
        mer2_r[...] = jnp.maximum(mer2_r[...], jnp.max(er2, axis=0, keepdims=True))


def _stage_b(acc, h, denp, b1, g1, be1, w2, al2_blk, ar2_blk):
    n = h.shape[0]
    grid = n // _N_BLK
    return pl.pallas_call(
        _body_b,
        grid=(2, grid),
        in_specs=[
            pl.BlockSpec((2, _N_BLK, 144), lambda p, i: (0, i, 0)),
            pl.BlockSpec((_N_BLK, 128), lambda p, i: (i, 0)),
            pl.BlockSpec((8, 128), lambda p, i: (0, 0)),
            pl.BlockSpec((1, 128), lambda p, i: (0, 0)),
            pl.BlockSpec((1, 128), lambda p, i: (0, 0)),
            pl.BlockSpec((1, 128), lambda p, i: (0, 0)),
            pl.BlockSpec((128, 128), lambda p, i: (0, 0)),
            pl.BlockSpec((128, 16), lambda p, i: (0, 0)),
            pl.BlockSpec((128, 16), lambda p, i: (0, 0)),
        ],
        out_specs=[
            pl.BlockSpec((_N_BLK, 128), lambda p, i: (i, 0)),
            pl.BlockSpec((_N_BLK, 144), lambda p, i: (i, 0)),
            pl.BlockSpec((_N_BLK, 16), lambda p, i: (i, 0)),
            pl.BlockSpec((1, 16), lambda p, i: (0, 0)),
            pl.BlockSpec((1, 16), lambda p, i: (0, 0)),
        ],
        out_shape=[
            jax.ShapeDtypeStruct((n, 128), jnp.float32),
            jax.ShapeDtypeStruct((n, 144), jnp.float32),
            jax.ShapeDtypeStruct((n, 16), jnp.float32),
            jax.ShapeDtypeStruct((1, 16), jnp.float32),
            jax.ShapeDtypeStruct((1, 16), jnp.float32),
        ],
        scratch_shapes=[pltpu.VMEM((2, 128), jnp.float32)],
    )(acc, h, denp, b1, g1, be1, w2, al2_blk, ar2_blk)


def _body_c(acc_r, h2_r, b2_r, g2_r, be2_r,
            wr0_r, br0_r, wr1_r, br1_r, wr2_r, br2_r,
            y_r, stats_r, hsum_r):
    p = pl.program_id(0)
    i = pl.program_id(1)
    asum = acc_r[0] + acc_r[1]
    den = lax.broadcast_in_dim(asum[:, 128:129], asum[:, :128].shape, (0, 1))
    den = jnp.where(den == 0.0, 1.0, den)
    rst = asum[:, :128] / den + b2_r[...]

    @pl.when(p == 0)
    def _():
        @pl.when(i == 0)
        def _():
            stats_r[...] = jnp.zeros_like(stats_r)

        stats_r[0:1] += jnp.sum(rst, axis=0, keepdims=True)
        stats_r[1:2] += jnp.sum(rst * rst, axis=0, keepdims=True)

    @pl.when(p == 1)
    def _():
        n_total = pl.num_programs(1) * rst.shape[0]
        mu = stats_r[0:1] / n_total
        var = stats_r[1:2] / n_total - mu * mu
        hbn = (rst - mu) * lax.rsqrt(var + 1e-5) * g2_r[...] + be2_r[...]
        h3 = h2_r[...] + _elu(hbn)

        @pl.when(i == 0)
        def _():
            hsum_r[...] = jnp.zeros_like(hsum_r)

        hsum_r[...] += jnp.sum(h3, axis=0, keepdims=True)

        @pl.when(i == pl.num_programs(1) - 1)
        def _():
            hg = hsum_r[...] / n_total
            y = jnp.maximum(jnp.dot(hg, wr0_r[...], preferred_element_type=jnp.float32)
                            + br0_r[...], 0.0)
            y = jnp.maximum(jnp.dot(y, wr1_r[...], preferred_element_type=jnp.float32)
                            + br1_r[...], 0.0)
            y = jnp.dot(y, wr2_r[...], preferred_element_type=jnp.float32) + br2_r[...]
            y_r[...] = jax.nn.sigmoid(y)


def _stage_c(acc, h2, b2, g2, be2, wr0, br0, wr1, br1, wr2, br2):
    n = h2.shape[0]
    grid = n // _N_BLK
    return pl.pallas_call(
        _body_c,
        grid=(2, grid),
        in_specs=[
            pl.BlockSpec((2, _N_BLK, 144), lambda p, i: (0, i, 0)),
            pl.BlockSpec((_N_BLK, 128), lambda p, i: (i, 0)),
            pl.BlockSpec((1, 128), lambda p, i: (0, 0)),
            pl.BlockSpec((1, 128), lambda p, i: (0, 0)),
            pl.BlockSpec((1, 128), lambda p, i: (0, 0)),
            pl.BlockSpec((128, 64), lambda p, i: (0, 0)),
            pl.BlockSpec((1, 64), lambda p, i: (0, 0)),
            pl.BlockSpec((64, 32), lambda p, i: (0, 0)),
            pl.BlockSpec((1, 32), lambda p, i: (0, 0)),
            pl.BlockSpec((32, 128), lambda p, i: (0, 0)),
            pl.BlockSpec((1, 128), lambda p, i: (0, 0)),
        ],
        out_specs=pl.BlockSpec((1, 128), lambda p, i: (0, 0)),
        out_shape=jax.ShapeDtypeStruct((1, 128), jnp.float32),
        scratch_shapes=[pltpu.VMEM((2, 128), jnp.float32),
                        pltpu.VMEM((1, 128), jnp.float32)],
    )(acc, h2, b2, g2, be2, wr0, br0, wr1, br1, wr2, br2)


_C = 80
_NW = 32
_DW = 144


def _edge_body(lane_map, n_pad, e_total,
               tbl_hbm, er_hbm, b_hbm, edge_hbm, out_hbm,
               idxs_v, idxd_v, rows0, rows1, err0, err1, b_v, acc_sh,
               sg0, sg1, ss0, ss1, si):
    import jax.experimental.pallas.tpu_sc as plsc

    cid = lax.axis_index("c")
    sid = lax.axis_index("s")
    wid = sid * 2 + cid
    nk = e_total // _C // _NW
    rows_per_tile = n_pad // 16
    zvec = jnp.zeros((16,), jnp.float32)

    def _zrow(i, _):
        for j in range(_DW // 16):
            rows0[i, pl.ds(16 * j, 16)] = zvec
        return 0

    lax.fori_loop(0, _C, _zrow, 0)
    for t in range(rows_per_tile // _C):
        pltpu.sync_copy(rows0,
                        acc_sh.at[pl.ds(sid * rows_per_tile + t * _C, _C), :])
    plsc.subcore_barrier()

    pltpu.sync_copy(b_hbm, b_v)
    bvec = b_v[0, :]
    bidx = [jnp.full((16, 1), lm, jnp.int32) for lm in range(8)]
    gdn = lax.GatherDimensionNumbers(offset_dims=(), collapsed_slice_dims=(0,),
                                     start_index_map=(0,))

    def _bcast(w, lm):
        return lax.gather(w, bidx[lm], gdn, (1,),
                          mode=lax.GatherScatterMode.PROMISE_IN_BOUNDS)

    bufs = ((rows0, err0, sg0, ss0), (rows1, err1, sg1, ss1))

    def _off(k):
        return (wid + k * _NW) * _C

    def _idx_issue(k, m):
        pltpu.async_copy(edge_hbm.at[0, pl.ds(_off(k), _C)], idxs_v.at[m], si)
        pltpu.async_copy(edge_hbm.at[1, pl.ds(_off(k), _C)], idxd_v.at[m], si)

    def _idx_wait(k, m):
        pltpu.make_async_copy(edge_hbm.at[0, pl.ds(_off(k), _C)],
                              idxs_v.at[m], si).wait()
        pltpu.make_async_copy(edge_hbm.at[1, pl.ds(_off(k), _C)],
                              idxd_v.at[m], si).wait()

    def _gather_issue(m, rows, err, sg):
        pltpu.async_copy(tbl_hbm.at[idxs_v.at[m]], rows, sg)
        pltpu.async_copy(er_hbm.at[idxd_v.at[m]], err, sg)

    def _gather_wait(m, rows, err, sg):
        pltpu.make_async_copy(tbl_hbm.at[idxs_v.at[m]], rows, sg).wait()
        pltpu.make_async_copy(er_hbm.at[idxd_v.at[m]], err, sg).wait()

    def _scatter_issue(m, rows, ss):
        pltpu.async_copy(rows, acc_sh.at[idxd_v.at[m]], ss, add=True)

    def _scatter_wait(m, rows, ss):
        pltpu.make_async_copy(rows, acc_sh.at[idxd_v.at[m]], ss).wait()

    def _compute(rows, err):
        def _edge(i, _):
            e0 = 2 * i
            e1 = 2 * i + 1
            t0 = rows[e0, pl.ds(128, 16)] + err[e0, :]
            t1 = rows[e1, pl.ds(128, 16)] + err[e1, :]
            w0 = jnp.exp(jnp.maximum(t0, 0.2 * t0) - bvec)
            w1 = jnp.exp(jnp.maximum(t1, 0.2 * t1) - bvec)
            rows[e0, pl.ds(128, 16)] = w0
            rows[e1, pl.ds(128, 16)] = w1
            lms = sorted(set(lane_map))
            wbs0 = {lm: _bcast(w0, lm) for lm in lms}
            wbs1 = {lm: _bcast(w1, lm) for lm in lms}
            for j in range(8):
                f0 = rows[e0, pl.ds(16 * j, 16)]
                f1 = rows[e1, pl.ds(16 * j, 16)]
                rows[e0, pl.ds(16 * j, 16)] = f0 * wbs0[lane_map[j]]
                rows[e1, pl.ds(16 * j, 16)] = f1 * wbs1[lane_map[j]]
            return 0

        lax.fori_loop(0, _C // 2, _edge, 0)

    pltpu.sync_copy(edge_hbm.at[0, pl.ds(_off(0), _C)], idxs_v.at[0])
    pltpu.sync_copy(edge_hbm.at[1, pl.ds(_off(0), _C)], idxd_v.at[0])
    _gather_issue(0, rows0, err0, sg0)
    _idx_issue(1, 1)

    def _iter(k, _):
        m = lax.bitwise_and(k, 3)
        m1 = lax.bitwise_and(k + 1, 3)
        m2 = lax.bitwise_and(k + 2, 3)
        for p in (0, 1):
            @pl.when(lax.bitwise_and(k, 1) == p)
            def _():
                cur_rows, cur_err, sgp, ssp = bufs[p]
                nxt_rows, nxt_err, sgn, ssn = bufs[1 - p]

                @pl.when(k < nk - 1)
                def _():
                    _idx_wait(k + 1, m1)

                    @pl.when(k >= 1)
                    def _():
                        _scatter_wait(m1, nxt_rows, ssn)

                    _gather_issue(m1, nxt_rows, nxt_err, sgn)

                @pl.when(k < nk - 2)
                def _():
                    _idx_issue(k + 2, m2)

                _gather_wait(m, cur_rows, cur_err, sgp)
                _compute(cur_rows, cur_err)
                _scatter_issue(m, cur_rows, ssp)
        return 0

    lax.fori_loop(0, nk, _iter, 0)
    _scatter_wait(0, bufs[(nk - 2) & 1][0], bufs[(nk - 2) & 1][3])
    _scatter_wait(0, bufs[(nk - 1) & 1][0], bufs[(nk - 1) & 1][3])
    plsc.subcore_barrier()

    for t in range(rows_per_tile // _C):
        r0 = sid * rows_per_tile + t * _C
        pltpu.sync_copy(acc_sh.at[pl.ds(r0, _C), :], rows0)
        pltpu.sync_copy(rows0, out_hbm.at[cid, pl.ds(r0, _C), :])


def _edge_phase_sc(tbl, er16, b16, edge_index, heads):
    import jax.experimental.pallas.tpu_sc as plsc

    n = tbl.shape[0]
    n_pad = ((n + 2047) // 2048) * 2048
    e_total = edge_index.shape[1]
    lane_map = tuple(j if heads == 8 else 0 for j in range(8))
    mesh = plsc.VectorSubcoreMesh(core_axis_name="c", subcore_axis_name="s",
                                  num_cores=2, num_subcores=16)
    body = functools.partial(_edge_body, lane_map, n_pad, e_total)
    return pl.kernel(
        body,
        out_type=jax.ShapeDtypeStruct((2, n_pad, _DW), jnp.float32),
        mesh=mesh,
        compiler_params=pltpu.CompilerParams(use_tc_tiling_on_sc=False,
                                             needs_layout_passes=False),
        scratch_types=[
            pltpu.VMEM((4, _C), jnp.int32),
            pltpu.VMEM((4, _C), jnp.int32),
            pltpu.VMEM((_C, _DW), jnp.float32),
            pltpu.VMEM((_C, _DW), jnp.float32),
            pltpu.VMEM((_C, 16), jnp.float32),
            pltpu.VMEM((_C, 16), jnp.float32),
            pltpu.VMEM((1, 16), jnp.float32),
            pltpu.VMEM_SHARED((n_pad, _DW), jnp.float32),
            pltpu.SemaphoreType.DMA,
            pltpu.SemaphoreType.DMA,
            pltpu.SemaphoreType.DMA,
            pltpu.SemaphoreType.DMA,
            pltpu.SemaphoreType.DMA,
        ],
    )(tbl, er16, b16.reshape(1, 16), edge_index)


def _edge_phase_jnp(tbl, er16, b16, src, dst, heads):
    n = tbl.shape[0]
    el = tbl[:, 128:144]
    w = jnp.exp(_lrelu(el[src] + er16[dst]) - b16[None, :])
    feat = tbl[:, :128]
    wexp = jnp.repeat(w[:, :heads], 128 // heads, axis=1)
    v = jnp.concatenate([feat[src] * wexp, w], axis=1)
    acc = jax.ops.segment_sum(v, dst, num_segments=n)
    return jnp.stack([acc, jnp.zeros_like(acc)])


def kernel(x, edge_index, W_in, b_in, W1, al1, ar1, b1, g1, be1,
           W2, al2, ar2, b2, g2, be2, Wr0, br0, Wr1, br1, Wr2, br2):
    heads, hid = al1.shape
    gat_out = al2.shape[1]

    eye = jnp.eye(heads, dtype=jnp.float32)
    al1_blk = (al1[:, :, None] * eye[:, None, :]).reshape(heads * hid, heads)
    al1_blk = jnp.pad(al1_blk, ((0, 0), (0, 16 - heads)))
    ar1_blk = (ar1[:, :, None] * eye[:, None, :]).reshape(heads * hid, heads)
    ar1_blk = jnp.pad(ar1_blk, ((0, 0), (0, 16 - heads)))
    al2_blk = jnp.pad(al2.T, ((0, 0), (0, 15)))
    ar2_blk = jnp.pad(ar2.T, ((0, 0), (0, 15)))
    denp = jnp.repeat(eye, hid, axis=1)

    row = lambda v: v.reshape(1, -1)
    src = edge_index[0]
    dst = edge_index[1]

    h, tbl1, er1, mel1, mer1 = _stage_a(x, W_in, row(b_in), W1, al1_blk, ar1_blk)
    b1v = _lrelu(mel1 + mer1)[0]
    acc1 = _edge_phase_sc(tbl1, er1, b1v, edge_index, heads)
    h2, tbl2, er2, mel2, mer2 = _stage_b(acc1, h, denp, row(b1), row(g1), row(be1),
                                         W2, al2_blk, ar2_blk)
    b2v = _lrelu(mel2 + mer2)[0]
    acc2 = _edge_phase_sc(tbl2, er2, b2v, edge_index, 1)
    return _stage_c(acc2, h2, row(b2), row(g2), row(be2),
                    Wr0, row(br0), Wr1, row(br1), Wr2, row(br2))


# speedup vs baseline: 102.8801x; 1.1363x over previous
import functools

import jax
import jax.numpy as jnp
from jax import lax
from jax.experimental import pallas as pl
from jax.experimental.pallas import tpu as pltpu

_N_BLK = 2000


def _elu(x):
    return jnp.where(x > 0, x, jnp.exp(x) - 1.0)


def _lrelu(x):
    return jnp.maximum(x, 0.2 * x)


def _body_a(x_r, win_r, bin_r, w1_r, al_r, ar_r,
            h_r, tbl_r, er_r, mel_r, mer_r):
    i = pl.program_id(0)
    h = jnp.dot(x_r[...], win_r[...], preferred_element_type=jnp.float32) + bin_r[...]
    feat = jnp.dot(h, w1_r[...], preferred_element_type=jnp.float32)
    el = jnp.dot(feat, al_r[...], preferred_element_type=jnp.float32)
    er = jnp.dot(feat, ar_r[...], preferred_element_type=jnp.float32)
    h_r[...] = h
    tbl_r[:, :128] = feat
    tbl_r[:, 128:144] = el
    er_r[...] = er

    @pl.when(i == 0)
    def _():
        mel_r[...] = jnp.full((1, 16), -1e30, jnp.float32)
        mer_r[...] = jnp.full((1, 16), -1e30, jnp.float32)

    mel_r[...] = jnp.maximum(mel_r[...], jnp.max(el, axis=0, keepdims=True))
    mer_r[...] = jnp.maximum(mer_r[...], jnp.max(er, axis=0, keepdims=True))


def _stage_a(x, w_in, b_in, w1, al_blk, ar_blk):
    n = x.shape[0]
    grid = n // _N_BLK
    return pl.pallas_call(
        _body_a,
        grid=(grid,),
        in_specs=[
            pl.BlockSpec((_N_BLK, 128), lambda i: (i, 0)),
            pl.BlockSpec((128, 128), lambda i: (0, 0)),
            pl.BlockSpec((1, 128), lambda i: (0, 0)),
            pl.BlockSpec((128, 128), lambda i: (0, 0)),
            pl.BlockSpec((128, 16), lambda i: (0, 0)),
            pl.BlockSpec((128, 16), lambda i: (0, 0)),
        ],
        out_specs=[
            pl.BlockSpec((_N_BLK, 128), lambda i: (i, 0)),
            pl.BlockSpec((_N_BLK, 144), lambda i: (i, 0)),
            pl.BlockSpec((_N_BLK, 16), lambda i: (i, 0)),
            pl.BlockSpec((1, 16), lambda i: (0, 0)),
            pl.BlockSpec((1, 16), lambda i: (0, 0)),
        ],
        out_shape=[
            jax.ShapeDtypeStruct((n, 128), jnp.float32),
            jax.ShapeDtypeStruct((n, 144), jnp.float32),
            jax.ShapeDtypeStruct((n, 16), jnp.float32),
            jax.ShapeDtypeStruct((1, 16), jnp.float32),
            jax.ShapeDtypeStruct((1, 16), jnp.float32),
        ],
    )(x, w_in, b_in, w1, al_blk, ar_blk)


def _body_b(acc_r, h_r, denp_r, b1_r, g1_r, be1_r, w2_r, al2_r, ar2_r,
            h2_r, tbl2_r, er2_r, mel2_r, mer2_r, stats_r):
    p = pl.program_id(0)
    i = pl.program_id(1)
    asum = acc_r[0] + acc_r[1]
    den = jnp.dot(asum[:, 128:136], denp_r[...],
                  preferred_element_type=jnp.float32)
    den = jnp.where(den == 0.0, 1.0, den)
    rst = asum[:, :128] / den + b1_r[...]

    @pl.when(p == 0)
    def _():
        @pl.when(i == 0)
        def _():
            stats_r[...] = jnp.zeros_like(stats_r)

        stats_r[0:1] += jnp.sum(rst, axis=0, keepdims=True)
        stats_r[1:2] += jnp.sum(rst * rst, axis=0, keepdims=True)

    @pl.when(p == 1)
    def _():
        n_total = pl.num_programs(1) * rst.shape[0]
        mu = stats_r[0:1] / n_total
        var = stats_r[1:2] / n_total - mu * mu
        hbn = (rst - mu) * lax.rsqrt(var + 1e-5) * g1_r[...] + be1_r[...]
        h2 = h_r[...] + _elu(hbn)
        h2_r[...] = h2
        feat2 = jnp.dot(h2, w2_r[...], preferred_element_type=jnp.float32)
        el2 = jnp.dot(feat2, al2_r[...], preferred_element_type=jnp.float32)
        er2 = jnp.dot(feat2, ar2_r[...], preferred_element_type=jnp.float32)
        tbl2_r[:, :128] = feat2
        tbl2_r[:, 128:144] = el2
        er2_r[...] = er2

        @pl.when(i == 0)
        def _():
            mel2_r[...] = jnp.full((1, 16), -1e30, jnp.float32)
            mer2_r[...] = jnp.full((1, 16), -1e30, jnp.float32)

        mel2_r[...] = jnp.maximum(mel2_r[...], jnp.max(el2, axis=0, keepdims=True))
        mer2_r[...] = jnp.maximum(mer2_r[...], jnp.max(er2, axis=0, keepdims=True))


def _stage_b(acc, h, denp, b1, g1, be1, w2, al2_blk, ar2_blk):
    n = h.shape[0]
    grid = n // _N_BLK
    return pl.pallas_call(
        _body_b,
        grid=(2, grid),
        in_specs=[
            pl.BlockSpec((2, _N_BLK, 144), lambda p, i: (0, i, 0)),
            pl.BlockSpec((_N_BLK, 128), lambda p, i: (i, 0)),
            pl.BlockSpec((8, 128), lambda p, i: (0, 0)),
            pl.BlockSpec((1, 128), lambda p, i: (0, 0)),
            pl.BlockSpec((1, 128), lambda p, i: (0, 0)),
            pl.BlockSpec((1, 128), lambda p, i: (0, 0)),
            pl.BlockSpec((128, 128), lambda p, i: (0, 0)),
            pl.BlockSpec((128, 16), lambda p, i: (0, 0)),
            pl.BlockSpec((128, 16), lambda p, i: (0, 0)),
        ],
        out_specs=[
            pl.BlockSpec((_N_BLK, 128), lambda p, i: (i, 0)),
            pl.BlockSpec((_N_BLK, 144), lambda p, i: (i, 0)),
            pl.BlockSpec((_N_BLK, 16), lambda p, i: (i, 0)),
            pl.BlockSpec((1, 16), lambda p, i: (0, 0)),
            pl.BlockSpec((1, 16), lambda p, i: (0, 0)),
        ],
        out_shape=[
            jax.ShapeDtypeStruct((n, 128), jnp.float32),
            jax.ShapeDtypeStruct((n, 144), jnp.float32),
            jax.ShapeDtypeStruct((n, 16), jnp.float32),
            jax.ShapeDtypeStruct((1, 16), jnp.float32),
            jax.ShapeDtypeStruct((1, 16), jnp.float32),
        ],
        scratch_shapes=[pltpu.VMEM((2, 128), jnp.float32)],
    )(acc, h, denp, b1, g1, be1, w2, al2_blk, ar2_blk)


def _body_c(acc_r, h2_r, b2_r, g2_r, be2_r,
            wr0_r, br0_r, wr1_r, br1_r, wr2_r, br2_r,
            y_r, stats_r, hsum_r):
    p = pl.program_id(0)
    i = pl.program_id(1)
    asum = acc_r[0] + acc_r[1]
    den = lax.broadcast_in_dim(asum[:, 128:129], asum[:, :128].shape, (0, 1))
    den = jnp.where(den == 0.0, 1.0, den)
    rst = asum[:, :128] / den + b2_r[...]

    @pl.when(p == 0)
    def _():
        @pl.when(i == 0)
        def _():
            stats_r[...] = jnp.zeros_like(stats_r)

        stats_r[0:1] += jnp.sum(rst, axis=0, keepdims=True)
        stats_r[1:2] += jnp.sum(rst * rst, axis=0, keepdims=True)

    @pl.when(p == 1)
    def _():
        n_total = pl.num_programs(1) * rst.shape[0]
        mu = stats_r[0:1] / n_total
        var = stats_r[1:2] / n_total - mu * mu
        hbn = (rst - mu) * lax.rsqrt(var + 1e-5) * g2_r[...] + be2_r[...]
        h3 = h2_r[...] + _elu(hbn)

        @pl.when(i == 0)
        def _():
            hsum_r[...] = jnp.zeros_like(hsum_r)

        hsum_r[...] += jnp.sum(h3, axis=0, keepdims=True)

        @pl.when(i == pl.num_programs(1) - 1)
        def _():
            hg = hsum_r[...] / n_total
            y = jnp.maximum(jnp.dot(hg, wr0_r[...], preferred_element_type=jnp.float32)
                            + br0_r[...], 0.0)
            y = jnp.maximum(jnp.dot(y, wr1_r[...], preferred_element_type=jnp.float32)
                            + br1_r[...], 0.0)
            y = jnp.dot(y, wr2_r[...], preferred_element_type=jnp.float32) + br2_r[...]
            y_r[...] = jax.nn.sigmoid(y)


def _stage_c(acc, h2, b2, g2, be2, wr0, br0, wr1, br1, wr2, br2):
    n = h2.shape[0]
    grid = n // _N_BLK
    return pl.pallas_call(
        _body_c,
        grid=(2, grid),
        in_specs=[
            pl.BlockSpec((2, _N_BLK, 144), lambda p, i: (0, i, 0)),
            pl.BlockSpec((_N_BLK, 128), lambda p, i: (i, 0)),
            pl.BlockSpec((1, 128), lambda p, i: (0, 0)),
            pl.BlockSpec((1, 128), lambda p, i: (0, 0)),
            pl.BlockSpec((1, 128), lambda p, i: (0, 0)),
            pl.BlockSpec((128, 64), lambda p, i: (0, 0)),
            pl.BlockSpec((1, 64), lambda p, i: (0, 0)),
            pl.BlockSpec((64, 32), lambda p, i: (0, 0)),
            pl.BlockSpec((1, 32), lambda p, i: (0, 0)),
            pl.BlockSpec((32, 128), lambda p, i: (0, 0)),
            pl.BlockSpec((1, 128), lambda p, i: (0, 0)),
        ],
        out_specs=pl.BlockSpec((1, 128), lambda p, i: (0, 0)),
        out_shape=jax.ShapeDtypeStruct((1, 128), jnp.float32),
        scratch_shapes=[pltpu.VMEM((2, 128), jnp.float32),
                        pltpu.VMEM((1, 128), jnp.float32)],
    )(acc, h2, b2, g2, be2, wr0, br0, wr1, br1, wr2, br2)


_C = 80
_NW = 32
_DW = 144


def _edge_body(lane_map, n_pad, e_total,
               tbl_hbm, er_hbm, b_hbm, edge_hbm, out_hbm,
               idxs_v, idxd_v, rows0, rows1, err0, err1, b_v, acc_sh,
               sg0, sg1, ss0, ss1, si):
    import jax.experimental.pallas.tpu_sc as plsc

    cid = lax.axis_index("c")
    sid = lax.axis_index("s")
    wid = sid * 2 + cid
    nk = e_total // _C // _NW
    rows_per_tile = n_pad // 16
    zvec = jnp.zeros((16,), jnp.float32)

    def _zrow(i, _):
        for j in range(_DW // 16):
            rows0[i, pl.ds(16 * j, 16)] = zvec
        return 0

    lax.fori_loop(0, _C, _zrow, 0)
    for t in range(rows_per_tile // _C):
        pltpu.sync_copy(rows0,
                        acc_sh.at[pl.ds(sid * rows_per_tile + t * _C, _C), :])
    plsc.subcore_barrier()

    pltpu.sync_copy(b_hbm, b_v)
    bvec = b_v[0, :]
    bidx = [jnp.full((16, 1), lm, jnp.int32) for lm in range(8)]
    gdn = lax.GatherDimensionNumbers(offset_dims=(), collapsed_slice_dims=(0,),
                                     start_index_map=(0,))

    def _bcast(w, lm):
        return lax.gather(w, bidx[lm], gdn, (1,),
                          mode=lax.GatherScatterMode.PROMISE_IN_BOUNDS)

    bufs = ((rows0, err0, sg0, ss0), (rows1, err1, sg1, ss1))

    def _off(k):
        return (wid + k * _NW) * _C

    def _idx_issue(k, m):
        pltpu.async_copy(edge_hbm.at[0, pl.ds(_off(k), _C)], idxs_v.at[m], si)
        pltpu.async_copy(edge_hbm.at[1, pl.ds(_off(k), _C)], idxd_v.at[m], si)

    def _idx_wait(k, m):
        pltpu.make_async_copy(edge_hbm.at[0, pl.ds(_off(k), _C)],
                              idxs_v.at[m], si).wait()
        pltpu.make_async_copy(edge_hbm.at[1, pl.ds(_off(k), _C)],
                              idxd_v.at[m], si).wait()

    def _gather_issue(m, rows, err, sg):
        pltpu.async_copy(tbl_hbm.at[idxs_v.at[m]], rows, sg)
        pltpu.async_copy(er_hbm.at[idxd_v.at[m]], err, sg)

    def _gather_wait(m, rows, err, sg):
        pltpu.make_async_copy(tbl_hbm.at[idxs_v.at[m]], rows, sg).wait()
        pltpu.make_async_copy(er_hbm.at[idxd_v.at[m]], err, sg).wait()

    def _scatter_issue(m, rows, ss):
        pltpu.async_copy(rows, acc_sh.at[idxd_v.at[m]], ss, add=True)

    def _scatter_wait(m, rows, ss):
        pltpu.make_async_copy(rows, acc_sh.at[idxd_v.at[m]], ss).wait()

    def _compute(rows, err):
        ilv = 4
        lms = sorted(set(lane_map))

        def _edge(i, _):
            es = [ilv * i + u for u in range(ilv)]
            ts = [rows[e, pl.ds(128, 16)] + err[e, :] for e in es]
            ws = [jnp.exp(jnp.maximum(t, 0.2 * t) - bvec) for t in ts]
            for e, w in zip(es, ws):
                rows[e, pl.ds(128, 16)] = w
            wbs = [{lm: _bcast(w, lm) for lm in lms} for w in ws]
            for j in range(8):
                fs = [rows[e, pl.ds(16 * j, 16)] for e in es]
                for u, e in enumerate(es):
                    rows[e, pl.ds(16 * j, 16)] = fs[u] * wbs[u][lane_map[j]]
            return 0

        lax.fori_loop(0, _C // ilv, _edge, 0)

    pltpu.sync_copy(edge_hbm.at[0, pl.ds(_off(0), _C)], idxs_v.at[0])
    pltpu.sync_copy(edge_hbm.at[1, pl.ds(_off(0), _C)], idxd_v.at[0])
    _gather_issue(0, rows0, err0, sg0)
    _idx_issue(1, 1)

    def _iter(k, _):
        m = lax.bitwise_and(k, 3)
        m1 = lax.bitwise_and(k + 1, 3)
        m2 = lax.bitwise_and(k + 2, 3)
        for p in (0, 1):
            @pl.when(lax.bitwise_and(k, 1) == p)
            def _():
                cur_rows, cur_err, sgp, ssp = bufs[p]
                nxt_rows, nxt_err, sgn, ssn = bufs[1 - p]

                @pl.when(k < nk - 1)
                def _():
                    _idx_wait(k + 1, m1)

                    @pl.when(k >= 1)
                    def _():
                        _scatter_wait(m1, nxt_rows, ssn)

                    _gather_issue(m1, nxt_rows, nxt_err, sgn)

                @pl.when(k < nk - 2)
                def _():
                    _idx_issue(k + 2, m2)

                _gather_wait(m, cur_rows, cur_err, sgp)
                _compute(cur_rows, cur_err)
                _scatter_issue(m, cur_rows, ssp)
        return 0

    lax.fori_loop(0, nk, _iter, 0)
    _scatter_wait(0, bufs[(nk - 2) & 1][0], bufs[(nk - 2) & 1][3])
    _scatter_wait(0, bufs[(nk - 1) & 1][0], bufs[(nk - 1) & 1][3])
    plsc.subcore_barrier()

    for t in range(rows_per_tile // _C):
        r0 = sid * rows_per_tile + t * _C
        pltpu.sync_copy(acc_sh.at[pl.ds(r0, _C), :], rows0)
        pltpu.sync_copy(rows0, out_hbm.at[cid, pl.ds(r0, _C), :])


def _edge_phase_sc(tbl, er16, b16, edge_index, heads):
    import jax.experimental.pallas.tpu_sc as plsc

    n = tbl.shape[0]
    n_pad = ((n + 2047) // 2048) * 2048
    e_total = edge_index.shape[1]
    lane_map = tuple(j if heads == 8 else 0 for j in range(8))
    mesh = plsc.VectorSubcoreMesh(core_axis_name="c", subcore_axis_name="s",
                                  num_cores=2, num_subcores=16)
    body = functools.partial(_edge_body, lane_map, n_pad, e_total)
    return pl.kernel(
        body,
        out_type=jax.ShapeDtypeStruct((2, n_pad, _DW), jnp.float32),
        mesh=mesh,
        compiler_params=pltpu.CompilerParams(use_tc_tiling_on_sc=False,
                                             needs_layout_passes=False),
        scratch_types=[
            pltpu.VMEM((4, _C), jnp.int32),
            pltpu.VMEM((4, _C), jnp.int32),
            pltpu.VMEM((_C, _DW), jnp.float32),
            pltpu.VMEM((_C, _DW), jnp.float32),
            pltpu.VMEM((_C, 16), jnp.float32),
            pltpu.VMEM((_C, 16), jnp.float32),
            pltpu.VMEM((1, 16), jnp.float32),
            pltpu.VMEM_SHARED((n_pad, _DW), jnp.float32),
            pltpu.SemaphoreType.DMA,
            pltpu.SemaphoreType.DMA,
            pltpu.SemaphoreType.DMA,
            pltpu.SemaphoreType.DMA,
            pltpu.SemaphoreType.DMA,
        ],
    )(tbl, er16, b16.reshape(1, 16), edge_index)


def _edge_phase_jnp(tbl, er16, b16, src, dst, heads):
    n = tbl.shape[0]
    el = tbl[:, 128:144]
    w = jnp.exp(_lrelu(el[src] + er16[dst]) - b16[None, :])
    feat = tbl[:, :128]
    wexp = jnp.repeat(w[:, :heads], 128 // heads, axis=1)
    v = jnp.concatenate([feat[src] * wexp, w], axis=1)
    acc = jax.ops.segment_sum(v, dst, num_segments=n)
    return jnp.stack([acc, jnp.zeros_like(acc)])


def kernel(x, edge_index, W_in, b_in, W1, al1, ar1, b1, g1, be1,
           W2, al2, ar2, b2, g2, be2, Wr0, br0, Wr1, br1, Wr2, br2):
    heads, hid = al1.shape
    gat_out = al2.shape[1]

    eye = jnp.eye(heads, dtype=jnp.float32)
    al1_blk = (al1[:, :, None] * eye[:, None, :]).reshape(heads * hid, heads)
    al1_blk = jnp.pad(al1_blk, ((0, 0), (0, 16 - heads)))
    ar1_blk = (ar1[:, :, None] * eye[:, None, :]).reshape(heads * hid, heads)
    ar1_blk = jnp.pad(ar1_blk, ((0, 0), (0, 16 - heads)))
    al2_blk = jnp.pad(al2.T, ((0, 0), (0, 15)))
    ar2_blk = jnp.pad(ar2.T, ((0, 0), (0, 15)))
    denp = jnp.repeat(eye, hid, axis=1)

    row = lambda v: v.reshape(1, -1)
    src = edge_index[0]
    dst = edge_index[1]

    h, tbl1, er1, mel1, mer1 = _stage_a(x, W_in, row(b_in), W1, al1_blk, ar1_blk)
    b1v = _lrelu(mel1 + mer1)[0]
    acc1 = _edge_phase_sc(tbl1, er1, b1v, edge_index, heads)
    h2, tbl2, er2, mel2, mer2 = _stage_b(acc1, h, denp, row(b1), row(g1), row(be1),
                                         W2, al2_blk, ar2_blk)
    b2v = _lrelu(mel2 + mer2)[0]
    acc2 = _edge_phase_sc(tbl2, er2, b2v, edge_index, 1)
    return _stage_c(acc2, h2, row(b2), row(g2), row(be2),
                    Wr0, row(br0), Wr1, row(br1), Wr2, row(br2))


# speedup vs baseline: 106.4263x; 1.0345x over previous
import functools

import jax
import jax.numpy as jnp
from jax import lax
from jax.experimental import pallas as pl
from jax.experimental.pallas import tpu as pltpu

_N_BLK = 2000


def _elu(x):
    return jnp.where(x > 0, x, jnp.exp(x) - 1.0)


def _lrelu(x):
    return jnp.maximum(x, 0.2 * x)


def _body_a(x_r, win_r, bin_r, w1_r, al_r, ar_r,
            h_r, tbl_r, er_r, mel_r, mer_r):
    i = pl.program_id(0)
    h = jnp.dot(x_r[...], win_r[...], preferred_element_type=jnp.float32) + bin_r[...]
    feat = jnp.dot(h, w1_r[...], preferred_element_type=jnp.float32)
    el = jnp.dot(feat, al_r[...], preferred_element_type=jnp.float32)
    er = jnp.dot(feat, ar_r[...], preferred_element_type=jnp.float32)
    h_r[...] = h
    tbl_r[:, :128] = feat
    tbl_r[:, 128:144] = el
    er_r[...] = er

    @pl.when(i == 0)
    def _():
        mel_r[...] = jnp.full((1, 16), -1e30, jnp.float32)
        mer_r[...] = jnp.full((1, 16), -1e30, jnp.float32)

    mel_r[...] = jnp.maximum(mel_r[...], jnp.max(el, axis=0, keepdims=True))
    mer_r[...] = jnp.maximum(mer_r[...], jnp.max(er, axis=0, keepdims=True))


def _stage_a(x, w_in, b_in, w1, al_blk, ar_blk):
    n = x.shape[0]
    grid = n // _N_BLK
    return pl.pallas_call(
        _body_a,
        grid=(grid,),
        in_specs=[
            pl.BlockSpec((_N_BLK, 128), lambda i: (i, 0)),
            pl.BlockSpec((128, 128), lambda i: (0, 0)),
            pl.BlockSpec((1, 128), lambda i: (0, 0)),
            pl.BlockSpec((128, 128), lambda i: (0, 0)),
            pl.BlockSpec((128, 16), lambda i: (0, 0)),
            pl.BlockSpec((128, 16), lambda i: (0, 0)),
        ],
        out_specs=[
            pl.BlockSpec((_N_BLK, 128), lambda i: (i, 0)),
            pl.BlockSpec((_N_BLK, 144), lambda i: (i, 0)),
            pl.BlockSpec((_N_BLK, 16), lambda i: (i, 0)),
            pl.BlockSpec((1, 16), lambda i: (0, 0)),
            pl.BlockSpec((1, 16), lambda i: (0, 0)),
        ],
        out_shape=[
            jax.ShapeDtypeStruct((n, 128), jnp.float32),
            jax.ShapeDtypeStruct((n, 144), jnp.float32),
            jax.ShapeDtypeStruct((n, 16), jnp.float32),
            jax.ShapeDtypeStruct((1, 16), jnp.float32),
            jax.ShapeDtypeStruct((1, 16), jnp.float32),
        ],
    )(x, w_in, b_in, w1, al_blk, ar_blk)


def _body_b(acc_r, h_r, denp_r, b1_r, g1_r, be1_r, w2_r, al2_r, ar2_r,
            h2_r, tbl2_r, er2_r, mel2_r, mer2_r, stats_r):
    p = pl.program_id(0)
    i = pl.program_id(1)
    asum = acc_r[0] + acc_r[1]
    den = jnp.dot(asum[:, 128:136], denp_r[...],
                  preferred_element_type=jnp.float32)
    den = jnp.where(den == 0.0, 1.0, den)
    rst = asum[:, :128] / den + b1_r[...]

    @pl.when(p == 0)
    def _():
        @pl.when(i == 0)
        def _():
            stats_r[...] = jnp.zeros_like(stats_r)

        stats_r[0:1] += jnp.sum(rst, axis=0, keepdims=True)
        stats_r[1:2] += jnp.sum(rst * rst, axis=0, keepdims=True)

    @pl.when(p == 1)
    def _():
        n_total = pl.num_programs(1) * rst.shape[0]
        mu = stats_r[0:1] / n_total
        var = stats_r[1:2] / n_total - mu * mu
        hbn = (rst - mu) * lax.rsqrt(var + 1e-5) * g1_r[...] + be1_r[...]
        h2 = h_r[...] + _elu(hbn)
        h2_r[...] = h2
        feat2 = jnp.dot(h2, w2_r[...], preferred_element_type=jnp.float32)
        el2 = jnp.dot(feat2, al2_r[...], preferred_element_type=jnp.float32)
        er2 = jnp.dot(feat2, ar2_r[...], preferred_element_type=jnp.float32)
        tbl2_r[:, :128] = feat2
        tbl2_r[:, 128:144] = el2
        er2_r[...] = er2

        @pl.when(i == 0)
        def _():
            mel2_r[...] = jnp.full((1, 16), -1e30, jnp.float32)
            mer2_r[...] = jnp.full((1, 16), -1e30, jnp.float32)

        mel2_r[...] = jnp.maximum(mel2_r[...], jnp.max(el2, axis=0, keepdims=True))
        mer2_r[...] = jnp.maximum(mer2_r[...], jnp.max(er2, axis=0, keepdims=True))


def _stage_b(acc, h, denp, b1, g1, be1, w2, al2_blk, ar2_blk):
    n = h.shape[0]
    grid = n // _N_BLK
    return pl.pallas_call(
        _body_b,
        grid=(2, grid),
        in_specs=[
            pl.BlockSpec((2, _N_BLK, 144), lambda p, i: (0, i, 0)),
            pl.BlockSpec((_N_BLK, 128), lambda p, i: (i, 0)),
            pl.BlockSpec((8, 128), lambda p, i: (0, 0)),
            pl.BlockSpec((1, 128), lambda p, i: (0, 0)),
            pl.BlockSpec((1, 128), lambda p, i: (0, 0)),
            pl.BlockSpec((1, 128), lambda p, i: (0, 0)),
            pl.BlockSpec((128, 128), lambda p, i: (0, 0)),
            pl.BlockSpec((128, 16), lambda p, i: (0, 0)),
            pl.BlockSpec((128, 16), lambda p, i: (0, 0)),
        ],
        out_specs=[
            pl.BlockSpec((_N_BLK, 128), lambda p, i: (i, 0)),
            pl.BlockSpec((_N_BLK, 144), lambda p, i: (i, 0)),
            pl.BlockSpec((_N_BLK, 16), lambda p, i: (i, 0)),
            pl.BlockSpec((1, 16), lambda p, i: (0, 0)),
            pl.BlockSpec((1, 16), lambda p, i: (0, 0)),
        ],
        out_shape=[
            jax.ShapeDtypeStruct((n, 128), jnp.float32),
            jax.ShapeDtypeStruct((n, 144), jnp.float32),
            jax.ShapeDtypeStruct((n, 16), jnp.float32),
            jax.ShapeDtypeStruct((1, 16), jnp.float32),
            jax.ShapeDtypeStruct((1, 16), jnp.float32),
        ],
        scratch_shapes=[pltpu.VMEM((2, 128), jnp.float32)],
    )(acc, h, denp, b1, g1, be1, w2, al2_blk, ar2_blk)


def _body_c(acc_r, h2_r, b2_r, g2_r, be2_r,
            wr0_r, br0_r, wr1_r, br1_r, wr2_r, br2_r,
            y_r, stats_r, hsum_r):
    p = pl.program_id(0)
    i = pl.program_id(1)
    asum = acc_r[0] + acc_r[1]
    den = lax.broadcast_in_dim(asum[:, 128:129], asum[:, :128].shape, (0, 1))
    den = jnp.where(den == 0.0, 1.0, den)
    rst = asum[:, :128] / den + b2_r[...]

    @pl.when(p == 0)
    def _():
        @pl.when(i == 0)
        def _():
            stats_r[...] = jnp.zeros_like(stats_r)

        stats_r[0:1] += jnp.sum(rst, axis=0, keepdims=True)
        stats_r[1:2] += jnp.sum(rst * rst, axis=0, keepdims=True)

    @pl.when(p == 1)
    def _():
        n_total = pl.num_programs(1) * rst.shape[0]
        mu = stats_r[0:1] / n_total
        var = stats_r[1:2] / n_total - mu * mu
        hbn = (rst - mu) * lax.rsqrt(var + 1e-5) * g2_r[...] + be2_r[...]
        h3 = h2_r[...] + _elu(hbn)

        @pl.when(i == 0)
        def _():
            hsum_r[...] = jnp.zeros_like(hsum_r)

        hsum_r[...] += jnp.sum(h3, axis=0, keepdims=True)

        @pl.when(i == pl.num_programs(1) - 1)
        def _():
            hg = hsum_r[...] / n_total
            y = jnp.maximum(jnp.dot(hg, wr0_r[...], preferred_element_type=jnp.float32)
                            + br0_r[...], 0.0)
            y = jnp.maximum(jnp.dot(y, wr1_r[...], preferred_element_type=jnp.float32)
                            + br1_r[...], 0.0)
            y = jnp.dot(y, wr2_r[...], preferred_element_type=jnp.float32) + br2_r[...]
            y_r[...] = jax.nn.sigmoid(y)


def _stage_c(acc, h2, b2, g2, be2, wr0, br0, wr1, br1, wr2, br2):
    n = h2.shape[0]
    grid = n // _N_BLK
    return pl.pallas_call(
        _body_c,
        grid=(2, grid),
        in_specs=[
            pl.BlockSpec((2, _N_BLK, 144), lambda p, i: (0, i, 0)),
            pl.BlockSpec((_N_BLK, 128), lambda p, i: (i, 0)),
            pl.BlockSpec((1, 128), lambda p, i: (0, 0)),
            pl.BlockSpec((1, 128), lambda p, i: (0, 0)),
            pl.BlockSpec((1, 128), lambda p, i: (0, 0)),
            pl.BlockSpec((128, 64), lambda p, i: (0, 0)),
            pl.BlockSpec((1, 64), lambda p, i: (0, 0)),
            pl.BlockSpec((64, 32), lambda p, i: (0, 0)),
            pl.BlockSpec((1, 32), lambda p, i: (0, 0)),
            pl.BlockSpec((32, 128), lambda p, i: (0, 0)),
            pl.BlockSpec((1, 128), lambda p, i: (0, 0)),
        ],
        out_specs=pl.BlockSpec((1, 128), lambda p, i: (0, 0)),
        out_shape=jax.ShapeDtypeStruct((1, 128), jnp.float32),
        scratch_shapes=[pltpu.VMEM((2, 128), jnp.float32),
                        pltpu.VMEM((1, 128), jnp.float32)],
    )(acc, h2, b2, g2, be2, wr0, br0, wr1, br1, wr2, br2)


_C = 80
_NW = 32
_DW = 144


def _edge_body(lane_map, n_pad, e_total,
               tbl_hbm, er_hbm, b_hbm, edge_hbm, out_hbm,
               idxs_v, idxd_v, rows0, rows1, err0, err1, b_v, acc_sh,
               sg0, sg1, ss0, ss1, si):
    import jax.experimental.pallas.tpu_sc as plsc

    cid = lax.axis_index("c")
    sid = lax.axis_index("s")
    wid = sid * 2 + cid
    nk = e_total // _C // _NW
    rows_per_tile = n_pad // 16
    zvec = jnp.zeros((16,), jnp.float32)

    def _zrow(i, _):
        for j in range(_DW // 16):
            rows0[i, pl.ds(16 * j, 16)] = zvec
        return 0

    lax.fori_loop(0, _C, _zrow, 0)
    for t in range(rows_per_tile // _C):
        pltpu.sync_copy(rows0,
                        acc_sh.at[pl.ds(sid * rows_per_tile + t * _C, _C), :])
    plsc.subcore_barrier()

    pltpu.sync_copy(b_hbm, b_v)
    bvec = b_v[0, :]
    bidx = [jnp.full((16, 1), lm, jnp.int32) for lm in range(8)]
    gdn = lax.GatherDimensionNumbers(offset_dims=(), collapsed_slice_dims=(0,),
                                     start_index_map=(0,))

    def _bcast(w, lm):
        return lax.gather(w, bidx[lm], gdn, (1,),
                          mode=lax.GatherScatterMode.PROMISE_IN_BOUNDS)

    bufs = ((rows0, err0, sg0, ss0), (rows1, err1, sg1, ss1))

    def _off(k):
        return (wid + k * _NW) * _C

    def _idx_issue(k, m):
        pltpu.async_copy(edge_hbm.at[0, pl.ds(_off(k), _C)], idxs_v.at[m], si)
        pltpu.async_copy(edge_hbm.at[1, pl.ds(_off(k), _C)], idxd_v.at[m], si)

    def _idx_wait(k, m):
        pltpu.make_async_copy(edge_hbm.at[0, pl.ds(_off(k), _C)],
                              idxs_v.at[m], si).wait()
        pltpu.make_async_copy(edge_hbm.at[1, pl.ds(_off(k), _C)],
                              idxd_v.at[m], si).wait()

    def _gather_issue(m, rows, err, sg):
        pltpu.async_copy(tbl_hbm.at[idxs_v.at[m]], rows, sg)
        pltpu.async_copy(er_hbm.at[idxd_v.at[m]], err, sg)

    def _gather_wait(m, rows, err, sg):
        pltpu.make_async_copy(tbl_hbm.at[idxs_v.at[m]], rows, sg).wait()
        pltpu.make_async_copy(er_hbm.at[idxd_v.at[m]], err, sg).wait()

    def _scatter_issue(m, rows, ss):
        pltpu.async_copy(rows, acc_sh.at[idxd_v.at[m]], ss, add=True)

    def _scatter_wait(m, rows, ss):
        pltpu.make_async_copy(rows, acc_sh.at[idxd_v.at[m]], ss).wait()

    def _compute(rows, err):
        ilv = 8
        lms = sorted(set(lane_map))

        def _edge(i, _):
            es = [ilv * i + u for u in range(ilv)]
            ts = [rows[e, pl.ds(128, 16)] + err[e, :] for e in es]
            ws = [jnp.exp(jnp.maximum(t, 0.2 * t) - bvec) for t in ts]
            for e, w in zip(es, ws):
                rows[e, pl.ds(128, 16)] = w
            wbs = [{lm: _bcast(w, lm) for lm in lms} for w in ws]
            for j in range(8):
                fs = [rows[e, pl.ds(16 * j, 16)] for e in es]
                for u, e in enumerate(es):
                    rows[e, pl.ds(16 * j, 16)] = fs[u] * wbs[u][lane_map[j]]
            return 0

        lax.fori_loop(0, _C // ilv, _edge, 0)

    pltpu.sync_copy(edge_hbm.at[0, pl.ds(_off(0), _C)], idxs_v.at[0])
    pltpu.sync_copy(edge_hbm.at[1, pl.ds(_off(0), _C)], idxd_v.at[0])
    _gather_issue(0, rows0, err0, sg0)
    _idx_issue(1, 1)

    def _iter(k, _):
        m = lax.bitwise_and(k, 3)
        m1 = lax.bitwise_and(k + 1, 3)
        m2 = lax.bitwise_and(k + 2, 3)
        for p in (0, 1):
            @pl.when(lax.bitwise_and(k, 1) == p)
            def _():
                cur_rows, cur_err, sgp, ssp = bufs[p]
                nxt_rows, nxt_err, sgn, ssn = bufs[1 - p]

                @pl.when(k < nk - 1)
                def _():
                    _idx_wait(k + 1, m1)

                    @pl.when(k >= 1)
                    def _():
                        _scatter_wait(m1, nxt_rows, ssn)

                    _gather_issue(m1, nxt_rows, nxt_err, sgn)

                @pl.when(k < nk - 2)
                def _():
                    _idx_issue(k + 2, m2)

                _gather_wait(m, cur_rows, cur_err, sgp)
                _compute(cur_rows, cur_err)
                _scatter_issue(m, cur_rows, ssp)
        return 0

    lax.fori_loop(0, nk, _iter, 0)
    _scatter_wait(0, bufs[(nk - 2) & 1][0], bufs[(nk - 2) & 1][3])
    _scatter_wait(0, bufs[(nk - 1) & 1][0], bufs[(nk - 1) & 1][3])
    plsc.subcore_barrier()

    for t in range(rows_per_tile // _C):
        r0 = sid * rows_per_tile + t * _C
        pltpu.sync_copy(acc_sh.at[pl.ds(r0, _C), :], rows0)
        pltpu.sync_copy(rows0, out_hbm.at[cid, pl.ds(r0, _C), :])


def _edge_phase_sc(tbl, er16, b16, edge_index, heads):
    import jax.experimental.pallas.tpu_sc as plsc

    n = tbl.shape[0]
    n_pad = ((n + 2047) // 2048) * 2048
    e_total = edge_index.shape[1]
    lane_map = tuple(j if heads == 8 else 0 for j in range(8))
    mesh = plsc.VectorSubcoreMesh(core_axis_name="c", subcore_axis_name="s",
                                  num_cores=2, num_subcores=16)
    body = functools.partial(_edge_body, lane_map, n_pad, e_total)
    return pl.kernel(
        body,
        out_type=jax.ShapeDtypeStruct((2, n_pad, _DW), jnp.float32),
        mesh=mesh,
        compiler_params=pltpu.CompilerParams(use_tc_tiling_on_sc=False,
                                             needs_layout_passes=False),
        scratch_types=[
            pltpu.VMEM((4, _C), jnp.int32),
            pltpu.VMEM((4, _C), jnp.int32),
            pltpu.VMEM((_C, _DW), jnp.float32),
            pltpu.VMEM((_C, _DW), jnp.float32),
            pltpu.VMEM((_C, 16), jnp.float32),
            pltpu.VMEM((_C, 16), jnp.float32),
            pltpu.VMEM((1, 16), jnp.float32),
            pltpu.VMEM_SHARED((n_pad, _DW), jnp.float32),
            pltpu.SemaphoreType.DMA,
            pltpu.SemaphoreType.DMA,
            pltpu.SemaphoreType.DMA,
            pltpu.SemaphoreType.DMA,
            pltpu.SemaphoreType.DMA,
        ],
    )(tbl, er16, b16.reshape(1, 16), edge_index)


def _edge_phase_jnp(tbl, er16, b16, src, dst, heads):
    n = tbl.shape[0]
    el = tbl[:, 128:144]
    w = jnp.exp(_lrelu(el[src] + er16[dst]) - b16[None, :])
    feat = tbl[:, :128]
    wexp = jnp.repeat(w[:, :heads], 128 // heads, axis=1)
    v = jnp.concatenate([feat[src] * wexp, w], axis=1)
    acc = jax.ops.segment_sum(v, dst, num_segments=n)
    return jnp.stack([acc, jnp.zeros_like(acc)])


def kernel(x, edge_index, W_in, b_in, W1, al1, ar1, b1, g1, be1,
           W2, al2, ar2, b2, g2, be2, Wr0, br0, Wr1, br1, Wr2, br2):
    heads, hid = al1.shape
    gat_out = al2.shape[1]

    eye = jnp.eye(heads, dtype=jnp.float32)
    al1_blk = (al1[:, :, None] * eye[:, None, :]).reshape(heads * hid, heads)
    al1_blk = jnp.pad(al1_blk, ((0, 0), (0, 16 - heads)))
    ar1_blk = (ar1[:, :, None] * eye[:, None, :]).reshape(heads * hid, heads)
    ar1_blk = jnp.pad(ar1_blk, ((0, 0), (0, 16 - heads)))
    al2_blk = jnp.pad(al2.T, ((0, 0), (0, 15)))
    ar2_blk = jnp.pad(ar2.T, ((0, 0), (0, 15)))
    denp = jnp.repeat(eye, hid, axis=1)

    row = lambda v: v.reshape(1, -1)
    src = edge_index[0]
    dst = edge_index[1]

    h, tbl1, er1, mel1, mer1 = _stage_a(x, W_in, row(b_in), W1, al1_blk, ar1_blk)
    b1v = _lrelu(mel1 + mer1)[0]
    acc1 = _edge_phase_sc(tbl1, er1, b1v, edge_index, heads)
    h2, tbl2, er2, mel2, mer2 = _stage_b(acc1, h, denp, row(b1), row(g1), row(be1),
                                         W2, al2_blk, ar2_blk)
    b2v = _lrelu(mel2 + mer2)[0]
    acc2 = _edge_phase_sc(tbl2, er2, b2v, edge_index, 1)
    return _stage_c(acc2, h2, row(b2), row(g2), row(be2),
                    Wr0, row(br0), Wr1, row(br1), Wr2, row(br2))


# speedup vs baseline: 108.9425x; 1.0236x over previous
import functools

import jax
import jax.numpy as jnp
from jax import lax
from jax.experimental import pallas as pl
from jax.experimental.pallas import tpu as pltpu

_N_BLK = 2000


def _elu(x):
    return jnp.where(x > 0, x, jnp.exp(x) - 1.0)


def _lrelu(x):
    return jnp.maximum(x, 0.2 * x)


def _body_a(x_r, win_r, bin_r, w1_r, al_r, ar_r,
            h_r, tbl_r, er_r, mel_r, mer_r):
    i = pl.program_id(0)
    h = jnp.dot(x_r[...], win_r[...], preferred_element_type=jnp.float32) + bin_r[...]
    feat = jnp.dot(h, w1_r[...], preferred_element_type=jnp.float32)
    el = jnp.dot(feat, al_r[...], preferred_element_type=jnp.float32)
    er = jnp.dot(feat, ar_r[...], preferred_element_type=jnp.float32)
    h_r[...] = h
    tbl_r[:, :128] = feat
    tbl_r[:, 128:144] = el
    er_r[...] = er

    @pl.when(i == 0)
    def _():
        mel_r[...] = jnp.full((1, 16), -1e30, jnp.float32)
        mer_r[...] = jnp.full((1, 16), -1e30, jnp.float32)

    mel_r[...] = jnp.maximum(mel_r[...], jnp.max(el, axis=0, keepdims=True))
    mer_r[...] = jnp.maximum(mer_r[...], jnp.max(er, axis=0, keepdims=True))


def _stage_a(x, w_in, b_in, w1, al_blk, ar_blk):
    n = x.shape[0]
    grid = n // _N_BLK
    return pl.pallas_call(
        _body_a,
        grid=(grid,),
        in_specs=[
            pl.BlockSpec((_N_BLK, 128), lambda i: (i, 0)),
            pl.BlockSpec((128, 128), lambda i: (0, 0)),
            pl.BlockSpec((1, 128), lambda i: (0, 0)),
            pl.BlockSpec((128, 128), lambda i: (0, 0)),
            pl.BlockSpec((128, 16), lambda i: (0, 0)),
            pl.BlockSpec((128, 16), lambda i: (0, 0)),
        ],
        out_specs=[
            pl.BlockSpec((_N_BLK, 128), lambda i: (i, 0)),
            pl.BlockSpec((_N_BLK, 144), lambda i: (i, 0)),
            pl.BlockSpec((_N_BLK, 16), lambda i: (i, 0)),
            pl.BlockSpec((1, 16), lambda i: (0, 0)),
            pl.BlockSpec((1, 16), lambda i: (0, 0)),
        ],
        out_shape=[
            jax.ShapeDtypeStruct((n, 128), jnp.float32),
            jax.ShapeDtypeStruct((n, 144), jnp.float32),
            jax.ShapeDtypeStruct((n, 16), jnp.float32),
            jax.ShapeDtypeStruct((1, 16), jnp.float32),
            jax.ShapeDtypeStruct((1, 16), jnp.float32),
        ],
    )(x, w_in, b_in, w1, al_blk, ar_blk)


def _body_b(acc_r, h_r, denp_r, b1_r, g1_r, be1_r, w2_r, al2_r, ar2_r,
            h2_r, tbl2_r, er2_r, mel2_r, mer2_r, stats_r, rst_r):
    p = pl.program_id(0)
    i = pl.program_id(1)

    @pl.when(p == 0)
    def _():
        asum = acc_r[0] + acc_r[1]
        den = jnp.dot(asum[:, 128:136], denp_r[...],
                      preferred_element_type=jnp.float32)
        den = jnp.where(den == 0.0, 1.0, den)
        rst = asum[:, :128] / den + b1_r[...]
        rst_r[pl.ds(i * rst.shape[0], rst.shape[0]), :] = rst

        @pl.when(i == 0)
        def _():
            stats_r[...] = jnp.zeros_like(stats_r)

        stats_r[0:1] += jnp.sum(rst, axis=0, keepdims=True)
        stats_r[1:2] += jnp.sum(rst * rst, axis=0, keepdims=True)

    @pl.when(p == 1)
    def _():
        blk = h_r.shape[0]
        rst = rst_r[pl.ds(i * blk, blk), :]
        n_total = pl.num_programs(1) * blk
        mu = stats_r[0:1] / n_total
        var = stats_r[1:2] / n_total - mu * mu
        hbn = (rst - mu) * lax.rsqrt(var + 1e-5) * g1_r[...] + be1_r[...]
        h2 = h_r[...] + _elu(hbn)
        h2_r[...] = h2
        feat2 = jnp.dot(h2, w2_r[...], preferred_element_type=jnp.float32)
        el2 = jnp.dot(feat2, al2_r[...], preferred_element_type=jnp.float32)
        er2 = jnp.dot(feat2, ar2_r[...], preferred_element_type=jnp.float32)
        tbl2_r[:, :128] = feat2
        tbl2_r[:, 128:144] = el2
        er2_r[...] = er2

        @pl.when(i == 0)
        def _():
            mel2_r[...] = jnp.full((1, 16), -1e30, jnp.float32)
            mer2_r[...] = jnp.full((1, 16), -1e30, jnp.float32)

        mel2_r[...] = jnp.maximum(mel2_r[...], jnp.max(el2, axis=0, keepdims=True))
        mer2_r[...] = jnp.maximum(mer2_r[...], jnp.max(er2, axis=0, keepdims=True))


def _stage_b(acc, h, denp, b1, g1, be1, w2, al2_blk, ar2_blk):
    n = h.shape[0]
    grid = n // _N_BLK
    return pl.pallas_call(
        _body_b,
        grid=(2, grid),
        in_specs=[
            pl.BlockSpec((2, _N_BLK, 144), lambda p, i: (0, i * (1 - p), 0)),
            pl.BlockSpec((_N_BLK, 128), lambda p, i: (i * p, 0)),
            pl.BlockSpec((8, 128), lambda p, i: (0, 0)),
            pl.BlockSpec((1, 128), lambda p, i: (0, 0)),
            pl.BlockSpec((1, 128), lambda p, i: (0, 0)),
            pl.BlockSpec((1, 128), lambda p, i: (0, 0)),
            pl.BlockSpec((128, 128), lambda p, i: (0, 0)),
            pl.BlockSpec((128, 16), lambda p, i: (0, 0)),
            pl.BlockSpec((128, 16), lambda p, i: (0, 0)),
        ],
        out_specs=[
            pl.BlockSpec((_N_BLK, 128), lambda p, i: (i, 0)),
            pl.BlockSpec((_N_BLK, 144), lambda p, i: (i, 0)),
            pl.BlockSpec((_N_BLK, 16), lambda p, i: (i, 0)),
            pl.BlockSpec((1, 16), lambda p, i: (0, 0)),
            pl.BlockSpec((1, 16), lambda p, i: (0, 0)),
        ],
        out_shape=[
            jax.ShapeDtypeStruct((n, 128), jnp.float32),
            jax.ShapeDtypeStruct((n, 144), jnp.float32),
            jax.ShapeDtypeStruct((n, 16), jnp.float32),
            jax.ShapeDtypeStruct((1, 16), jnp.float32),
            jax.ShapeDtypeStruct((1, 16), jnp.float32),
        ],
        scratch_shapes=[pltpu.VMEM((2, 128), jnp.float32),
                        pltpu.VMEM((n, 128), jnp.float32)],
    )(acc, h, denp, b1, g1, be1, w2, al2_blk, ar2_blk)


def _body_c(acc_r, h2_r, b2_r, g2_r, be2_r,
            wr0_r, br0_r, wr1_r, br1_r, wr2_r, br2_r,
            y_r, stats_r, hsum_r, rst_r):
    p = pl.program_id(0)
    i = pl.program_id(1)

    @pl.when(p == 0)
    def _():
        asum = acc_r[0] + acc_r[1]
        den = lax.broadcast_in_dim(asum[:, 128:129], asum[:, :128].shape,
                                   (0, 1))
        den = jnp.where(den == 0.0, 1.0, den)
        rst = asum[:, :128] / den + b2_r[...]
        rst_r[pl.ds(i * rst.shape[0], rst.shape[0]), :] = rst

        @pl.when(i == 0)
        def _():
            stats_r[...] = jnp.zeros_like(stats_r)

        stats_r[0:1] += jnp.sum(rst, axis=0, keepdims=True)
        stats_r[1:2] += jnp.sum(rst * rst, axis=0, keepdims=True)

    @pl.when(p == 1)
    def _():
        blk = h2_r.shape[0]
        rst = rst_r[pl.ds(i * blk, blk), :]
        n_total = pl.num_programs(1) * blk
        mu = stats_r[0:1] / n_total
        var = stats_r[1:2] / n_total - mu * mu
        hbn = (rst - mu) * lax.rsqrt(var + 1e-5) * g2_r[...] + be2_r[...]
        h3 = h2_r[...] + _elu(hbn)

        @pl.when(i == 0)
        def _():
            hsum_r[...] = jnp.zeros_like(hsum_r)

        hsum_r[...] += jnp.sum(h3, axis=0, keepdims=True)

        @pl.when(i == pl.num_programs(1) - 1)
        def _():
            hg = hsum_r[...] / n_total
            y = jnp.maximum(jnp.dot(hg, wr0_r[...], preferred_element_type=jnp.float32)
                            + br0_r[...], 0.0)
            y = jnp.maximum(jnp.dot(y, wr1_r[...], preferred_element_type=jnp.float32)
                            + br1_r[...], 0.0)
            y = jnp.dot(y, wr2_r[...], preferred_element_type=jnp.float32) + br2_r[...]
            y_r[...] = jax.nn.sigmoid(y)


def _stage_c(acc, h2, b2, g2, be2, wr0, br0, wr1, br1, wr2, br2):
    n = h2.shape[0]
    grid = n // _N_BLK
    return pl.pallas_call(
        _body_c,
        grid=(2, grid),
        in_specs=[
            pl.BlockSpec((2, _N_BLK, 144), lambda p, i: (0, i * (1 - p), 0)),
            pl.BlockSpec((_N_BLK, 128), lambda p, i: (i * p, 0)),
            pl.BlockSpec((1, 128), lambda p, i: (0, 0)),
            pl.BlockSpec((1, 128), lambda p, i: (0, 0)),
            pl.BlockSpec((1, 128), lambda p, i: (0, 0)),
            pl.BlockSpec((128, 64), lambda p, i: (0, 0)),
            pl.BlockSpec((1, 64), lambda p, i: (0, 0)),
            pl.BlockSpec((64, 32), lambda p, i: (0, 0)),
            pl.BlockSpec((1, 32), lambda p, i: (0, 0)),
            pl.BlockSpec((32, 128), lambda p, i: (0, 0)),
            pl.BlockSpec((1, 128), lambda p, i: (0, 0)),
        ],
        out_specs=pl.BlockSpec((1, 128), lambda p, i: (0, 0)),
        out_shape=jax.ShapeDtypeStruct((1, 128), jnp.float32),
        scratch_shapes=[pltpu.VMEM((2, 128), jnp.float32),
                        pltpu.VMEM((1, 128), jnp.float32),
                        pltpu.VMEM((n, 128), jnp.float32)],
    )(acc, h2, b2, g2, be2, wr0, br0, wr1, br1, wr2, br2)


_C = 80
_NW = 32
_DW = 144


def _edge_body(lane_map, n_pad, e_total,
               tbl_hbm, er_hbm, b_hbm, edge_hbm, out_hbm,
               idxs_v, idxd_v, rows0, rows1, err0, err1, b_v, acc_sh,
               sg0, sg1, ss0, ss1, si):
    import jax.experimental.pallas.tpu_sc as plsc

    cid = lax.axis_index("c")
    sid = lax.axis_index("s")
    wid = sid * 2 + cid
    nk = e_total // _C // _NW
    rows_per_tile = n_pad // 16
    zvec = jnp.zeros((16,), jnp.float32)

    def _zrow(i, _):
        for j in range(_DW // 16):
            rows0[i, pl.ds(16 * j, 16)] = zvec
        return 0

    lax.fori_loop(0, _C, _zrow, 0)
    for t in range(rows_per_tile // _C):
        pltpu.sync_copy(rows0,
                        acc_sh.at[pl.ds(sid * rows_per_tile + t * _C, _C), :])
    plsc.subcore_barrier()

    pltpu.sync_copy(b_hbm, b_v)
    bvec = b_v[0, :]
    bidx = [jnp.full((16, 1), lm, jnp.int32) for lm in range(8)]
    gdn = lax.GatherDimensionNumbers(offset_dims=(), collapsed_slice_dims=(0,),
                                     start_index_map=(0,))

    def _bcast(w, lm):
        return lax.gather(w, bidx[lm], gdn, (1,),
                          mode=lax.GatherScatterMode.PROMISE_IN_BOUNDS)

    bufs = ((rows0, err0, sg0, ss0), (rows1, err1, sg1, ss1))

    def _off(k):
        return (wid + k * _NW) * _C

    def _idx_issue(k, m):
        pltpu.async_copy(edge_hbm.at[0, pl.ds(_off(k), _C)], idxs_v.at[m], si)
        pltpu.async_copy(edge_hbm.at[1, pl.ds(_off(k), _C)], idxd_v.at[m], si)

    def _idx_wait(k, m):
        pltpu.make_async_copy(edge_hbm.at[0, pl.ds(_off(k), _C)],
                              idxs_v.at[m], si).wait()
        pltpu.make_async_copy(edge_hbm.at[1, pl.ds(_off(k), _C)],
                              idxd_v.at[m], si).wait()

    def _gather_issue(m, rows, err, sg):
        pltpu.async_copy(tbl_hbm.at[idxs_v.at[m]], rows, sg)
        pltpu.async_copy(er_hbm.at[idxd_v.at[m]], err, sg)

    def _gather_wait(m, rows, err, sg):
        pltpu.make_async_copy(tbl_hbm.at[idxs_v.at[m]], rows, sg).wait()
        pltpu.make_async_copy(er_hbm.at[idxd_v.at[m]], err, sg).wait()

    def _scatter_issue(m, rows, ss):
        pltpu.async_copy(rows, acc_sh.at[idxd_v.at[m]], ss, add=True)

    def _scatter_wait(m, rows, ss):
        pltpu.make_async_copy(rows, acc_sh.at[idxd_v.at[m]], ss).wait()

    def _compute(rows, err):
        ilv = 8
        lms = sorted(set(lane_map))

        def _edge(i, _):
            es = [ilv * i + u for u in range(ilv)]
            ts = [rows[e, pl.ds(128, 16)] + err[e, :] for e in es]
            ws = [jnp.exp(jnp.maximum(t, 0.2 * t) - bvec) for t in ts]
            for e, w in zip(es, ws):
                rows[e, pl.ds(128, 16)] = w
            wbs = [{lm: _bcast(w, lm) for lm in lms} for w in ws]
            for j in range(8):
                fs = [rows[e, pl.ds(16 * j, 16)] for e in es]
                for u, e in enumerate(es):
                    rows[e, pl.ds(16 * j, 16)] = fs[u] * wbs[u][lane_map[j]]
            return 0

        lax.fori_loop(0, _C // ilv, _edge, 0)

    pltpu.sync_copy(edge_hbm.at[0, pl.ds(_off(0), _C)], idxs_v.at[0])
    pltpu.sync_copy(edge_hbm.at[1, pl.ds(_off(0), _C)], idxd_v.at[0])
    _gather_issue(0, rows0, err0, sg0)
    _idx_issue(1, 1)

    def _iter(k, _):
        m = lax.bitwise_and(k, 3)
        m1 = lax.bitwise_and(k + 1, 3)
        m2 = lax.bitwise_and(k + 2, 3)
        for p in (0, 1):
            @pl.when(lax.bitwise_and(k, 1) == p)
            def _():
                cur_rows, cur_err, sgp, ssp = bufs[p]
                nxt_rows, nxt_err, sgn, ssn = bufs[1 - p]

                @pl.when(k < nk - 1)
                def _():
                    _idx_wait(k + 1, m1)

                    @pl.when(k >= 1)
                    def _():
                        _scatter_wait(m1, nxt_rows, ssn)

                    _gather_issue(m1, nxt_rows, nxt_err, sgn)

                @pl.when(k < nk - 2)
                def _():
                    _idx_issue(k + 2, m2)

                _gather_wait(m, cur_rows, cur_err, sgp)
                _compute(cur_rows, cur_err)
                _scatter_issue(m, cur_rows, ssp)
        return 0

    lax.fori_loop(0, nk, _iter, 0)
    _scatter_wait(0, bufs[(nk - 2) & 1][0], bufs[(nk - 2) & 1][3])
    _scatter_wait(0, bufs[(nk - 1) & 1][0], bufs[(nk - 1) & 1][3])
    plsc.subcore_barrier()

    for t in range(rows_per_tile // _C):
        r0 = sid * rows_per_tile + t * _C
        pltpu.sync_copy(acc_sh.at[pl.ds(r0, _C), :], rows0)
        pltpu.sync_copy(rows0, out_hbm.at[cid, pl.ds(r0, _C), :])


def _edge_phase_sc(tbl, er16, b16, edge_index, heads):
    import jax.experimental.pallas.tpu_sc as plsc

    n = tbl.shape[0]
    n_pad = ((n + 2047) // 2048) * 2048
    e_total = edge_index.shape[1]
    lane_map = tuple(j if heads == 8 else 0 for j in range(8))
    mesh = plsc.VectorSubcoreMesh(core_axis_name="c", subcore_axis_name="s",
                                  num_cores=2, num_subcores=16)
    body = functools.partial(_edge_body, lane_map, n_pad, e_total)
    return pl.kernel(
        body,
        out_type=jax.ShapeDtypeStruct((2, n_pad, _DW), jnp.float32),
        mesh=mesh,
        compiler_params=pltpu.CompilerParams(use_tc_tiling_on_sc=False,
                                             needs_layout_passes=False),
        scratch_types=[
            pltpu.VMEM((4, _C), jnp.int32),
            pltpu.VMEM((4, _C), jnp.int32),
            pltpu.VMEM((_C, _DW), jnp.float32),
            pltpu.VMEM((_C, _DW), jnp.float32),
            pltpu.VMEM((_C, 16), jnp.float32),
            pltpu.VMEM((_C, 16), jnp.float32),
            pltpu.VMEM((1, 16), jnp.float32),
            pltpu.VMEM_SHARED((n_pad, _DW), jnp.float32),
            pltpu.SemaphoreType.DMA,
            pltpu.SemaphoreType.DMA,
            pltpu.SemaphoreType.DMA,
            pltpu.SemaphoreType.DMA,
            pltpu.SemaphoreType.DMA,
        ],
    )(tbl, er16, b16.reshape(1, 16), edge_index)


def _edge_phase_jnp(tbl, er16, b16, src, dst, heads):
    n = tbl.shape[0]
    el = tbl[:, 128:144]
    w = jnp.exp(_lrelu(el[src] + er16[dst]) - b16[None, :])
    feat = tbl[:, :128]
    wexp = jnp.repeat(w[:, :heads], 128 // heads, axis=1)
    v = jnp.concatenate([feat[src] * wexp, w], axis=1)
    acc = jax.ops.segment_sum(v, dst, num_segments=n)
    return jnp.stack([acc, jnp.zeros_like(acc)])


def kernel(x, edge_index, W_in, b_in, W1, al1, ar1, b1, g1, be1,
           W2, al2, ar2, b2, g2, be2, Wr0, br0, Wr1, br1, Wr2, br2):
    heads, hid = al1.shape
    gat_out = al2.shape[1]

    eye = jnp.eye(heads, dtype=jnp.float32)
    al1_blk = (al1[:, :, None] * eye[:, None, :]).reshape(heads * hid, heads)
    al1_blk = jnp.pad(al1_blk, ((0, 0), (0, 16 - heads)))
    ar1_blk = (ar1[:, :, None] * eye[:, None, :]).reshape(heads * hid, heads)
    ar1_blk = jnp.pad(ar1_blk, ((0, 0), (0, 16 - heads)))
    al2_blk = jnp.pad(al2.T, ((0, 0), (0, 15)))
    ar2_blk = jnp.pad(ar2.T, ((0, 0), (0, 15)))
    denp = jnp.repeat(eye, hid, axis=1)

    row = lambda v: v.reshape(1, -1)
    src = edge_index[0]
    dst = edge_index[1]

    h, tbl1, er1, mel1, mer1 = _stage_a(x, W_in, row(b_in), W1, al1_blk, ar1_blk)
    b1v = _lrelu(mel1 + mer1)[0]
    acc1 = _edge_phase_sc(tbl1, er1, b1v, edge_index, heads)
    h2, tbl2, er2, mel2, mer2 = _stage_b(acc1, h, denp, row(b1), row(g1), row(be1),
                                         W2, al2_blk, ar2_blk)
    b2v = _lrelu(mel2 + mer2)[0]
    acc2 = _edge_phase_sc(tbl2, er2, b2v, edge_index, 1)
    return _stage_c(acc2, h2, row(b2), row(g2), row(be2),
                    Wr0, row(br0), Wr1, row(br1), Wr2, row(br2))
